# baseline (speedup 1.0000x reference)
import functools

import jax
import jax.numpy as jnp
from jax import lax
from jax.experimental import pallas as pl
from jax.experimental.pallas import tpu as pltpu
from jax.experimental.pallas import tpu_sc as plsc

N = 10000
E = 320000
D = 128
H = 4
DH = 32
ED = 16
EF = D + ED
HB = max(32, EF // 2)
FF = 4 * D
EUH = max(EF, D)

NC = 2
NS = 16
NW = NC * NS
EPW = E // NW
G = 80
NG = EPW // G

BE = 2560
BN = 2000


def _sc_gather(source, target, sidx2, tidx2):
    mesh = plsc.VectorSubcoreMesh(core_axis_name="c", subcore_axis_name="s")

    @functools.partial(
        pl.kernel,
        out_type=(
            jax.ShapeDtypeStruct((E, D), jnp.float32),
            jax.ShapeDtypeStruct((E, D), jnp.float32),
        ),
        mesh=mesh,
        scratch_types=[
            pltpu.VMEM((NG, G), jnp.int32),
            pltpu.VMEM((NG, G), jnp.int32),
            pltpu.VMEM((2, G, D), jnp.float32),
            pltpu.VMEM((2, G, D), jnp.float32),
            pltpu.SemaphoreType.DMA,
            pltpu.SemaphoreType.DMA,
            pltpu.SemaphoreType.DMA,
            pltpu.SemaphoreType.DMA,
            pltpu.SemaphoreType.DMA,
            pltpu.SemaphoreType.DMA,
            pltpu.SemaphoreType.DMA,
            pltpu.SemaphoreType.DMA,
        ],
    )
    def k(src_hbm, tgt_hbm, sidx_hbm, tidx_hbm, outS, outT,
          sidx_v, tidx_v, rowS, rowT, sS0, sS1, sT0, sT1, w0, w1, w2, w3):
        wid = lax.axis_index("s") * NC + lax.axis_index("c")
        base = wid * EPW
        pltpu.sync_copy(sidx_hbm.at[wid], sidx_v)
        pltpu.sync_copy(tidx_hbm.at[wid], tidx_v)

        def pair(j0, j1):
            gS0 = pltpu.async_copy(src_hbm.at[sidx_v.at[j0]], rowS.at[0], sS0)
            gT0 = pltpu.async_copy(tgt_hbm.at[tidx_v.at[j0]], rowT.at[0], sT0)
            gS1 = pltpu.async_copy(src_hbm.at[sidx_v.at[j1]], rowS.at[1], sS1)
            gT1 = pltpu.async_copy(tgt_hbm.at[tidx_v.at[j1]], rowT.at[1], sT1)
            gS0.wait()
            wS0 = pltpu.async_copy(rowS.at[0], outS.at[pl.ds(base + j0 * G, G)], w0)
            gT0.wait()
            wT0 = pltpu.async_copy(rowT.at[0], outT.at[pl.ds(base + j0 * G, G)], w1)
            gS1.wait()
            wS1 = pltpu.async_copy(rowS.at[1], outS.at[pl.ds(base + j1 * G, G)], w2)
            gT1.wait()
            wT1 = pltpu.async_copy(rowT.at[1], outT.at[pl.ds(base + j1 * G, G)], w3)
            wS0.wait()
            wT0.wait()
            wS1.wait()
            wT1.wait()

        def body(jj, carry):
            pair(jj * 2, jj * 2 + 1)
            return carry

        lax.fori_loop(0, NG // 2, body, 0)
        if NG % 2:
            j = NG - 1
            gS0 = pltpu.async_copy(src_hbm.at[sidx_v.at[j]], rowS.at[0], sS0)
            gT0 = pltpu.async_copy(tgt_hbm.at[tidx_v.at[j]], rowT.at[0], sT0)
            gS0.wait()
            pltpu.sync_copy(rowS.at[0], outS.at[pl.ds(base + j * G, G)])
            gT0.wait()
            pltpu.sync_copy(rowT.at[0], outT.at[pl.ds(base + j * G, G)])

    return k(source, target, sidx2, tidx2)


HALF = N // NC
ACC_ROWS = HALF + 8
EPT = E // NS
NGS = EPT // G


def _make_sc_scatter(W, tc_tiling):
    mesh = plsc.VectorSubcoreMesh(core_axis_name="c", subcore_axis_name="s")

    @functools.partial(
        pl.kernel,
        out_type=jax.ShapeDtypeStruct((N, W), jnp.float32),
        mesh=mesh,
        scratch_types=[
            pltpu.VMEM((2, G), jnp.int32),
            pltpu.VMEM((2, G), jnp.int32),
            pltpu.VMEM((2, G, W), jnp.float32),
            pltpu.VMEM_SHARED((ACC_ROWS, W), jnp.float32),
            pltpu.SemaphoreType.DMA,
            pltpu.SemaphoreType.DMA,
            pltpu.SemaphoreType.DMA,
            pltpu.SemaphoreType.DMA,
        ],
        compiler_params=pltpu.CompilerParams(use_tc_tiling_on_sc=tc_tiling),
    )
    def k(pay_hbm, tidx_hbm, zP_hbm, outP,
          idx_v, map_v, rowP, accP, sI0, sI1, sP0, sP1):
        cid = lax.axis_index("c")
        sid = lax.axis_index("s")
        tbase = sid * EPT
        nbase = cid * HALF

        @pl.when(sid == 0)
        def _init():
            pltpu.sync_copy(zP_hbm, accP)

        plsc.subcore_barrier()

        def chunk(j, slot, sI, sP):
            lI = pltpu.async_copy(tidx_hbm.at[pl.ds(tbase + j * G, G)],
                                  idx_v.at[slot], sI)
            lP = pltpu.async_copy(pay_hbm.at[pl.ds(tbase + j * G, G)],
                                  rowP.at[slot], sP)
            lI.wait()
            for kk in range(G // 16):
                v = idx_v[slot, pl.ds(kk * 16, 16)]
                loc = v - nbase
                inb = (loc >= 0) & (loc < HALF)
                map_v[slot, pl.ds(kk * 16, 16)] = jnp.where(inb, loc, HALF)
            lP.wait()
            pltpu.sync_copy(rowP.at[slot], accP.at[map_v.at[slot]], add=True)

        def body(jj, carry):
            chunk(jj * 2, 0, sI0, sP0)
            chunk(jj * 2 + 1, 1, sI1, sP1)
            return carry

        lax.fori_loop(0, NGS // 2, body, 0)
        plsc.subcore_barrier()

        @pl.when(sid == 0)
        def _dump():
            pltpu.sync_copy(accP.at[pl.ds(0, HALF)],
                            outP.at[pl.ds(cid * HALF, HALF)])

    return k


def _ln_rows(x, eps=1e-5):
    mu = jnp.mean(x, axis=-1, keepdims=True)
    v = jnp.mean(x * x, axis=-1, keepdims=True) - mu * mu
    return (x - mu) * lax.rsqrt(v + eps)


def _edge_body(S_ref, T_ref, EAT_ref,
               Wkv_ref, Wq_ref, Wen_ref, Wee_ref, be_ref,
               Wb1n_ref, Wb1e_ref, bb1_ref, Wb2_ref, bb2_ref,
               Wu1a_ref, Wu1b_ref, Wu1c_ref, bu1_ref, Wu2_ref, bu2_ref,
               gkv_ref, bkv_ref, gq_ref, bq_ref, gen_ref, ben_ref,
               it_ref, sel_ref, selt_ref,
               num_ref, ext_ref, updt_ref):
    f32 = jnp.float32
    s = S_ref[...]
    t = T_ref[...]
    ea = EAT_ref[...].T
    sn = _ln_rows(s) * gkv_ref[...] + bkv_ref[...]
    tn = _ln_rows(t) * gq_ref[...] + bq_ref[...]
    kv = jnp.dot(sn, Wkv_ref[...], preferred_element_type=f32)
    k_n = kv[:, :D]
    v_n = kv[:, D:]
    q = jnp.dot(tn, Wq_ref[...], preferred_element_type=f32)
    pw = s * t
    ssum = jnp.sum(pw, axis=-1, keepdims=True) + jnp.sum(ea, axis=-1, keepdims=True)
    ssq = jnp.sum(pw * pw, axis=-1, keepdims=True) + jnp.sum(ea * ea, axis=-1, keepdims=True)
    mu = ssum / EF
    var = ssq / EF - mu * mu
    inv = lax.rsqrt(var + 1e-5)
    efn = (pw - mu) * inv * gen_ref[:, :D] + ben_ref[:, :D]
    efe = (ea - mu) * inv * gen_ref[:, D:] + ben_ref[:, D:]
    kve = (jnp.dot(efn, Wen_ref[...], preferred_element_type=f32)
           + jnp.dot(efe, Wee_ref[...], preferred_element_type=f32)
           + be_ref[...])
    sk = k_n + kve[:, :D]
    sv = v_n + kve[:, D:]
    h = jax.nn.relu(jnp.dot(efn, Wb1n_ref[...], preferred_element_type=f32)
                    + jnp.dot(efe, Wb1e_ref[...], preferred_element_type=f32)
                    + bb1_ref[...])
    bias = jnp.dot(h, Wb2_ref[...], preferred_element_type=f32) + bb2_ref[...]
    prod = q * sk
    logits = (jnp.dot(prod, sel_ref[...], preferred_element_type=f32)
              * it_ref[...] + bias)
    ex = jnp.exp(logits)
    exb = jnp.dot(ex, selt_ref[...], preferred_element_type=f32)
    num_ref[...] = exb * sv
    ext_ref[...] = ex.T
    g1 = jax.nn.relu(jnp.dot(efn, Wu1a_ref[...], preferred_element_type=f32)
                     + jnp.dot(efe, Wu1b_ref[...], preferred_element_type=f32)
                     + jnp.dot(sv, Wu1c_ref[...], preferred_element_type=f32)
                     + bu1_ref[...])
    updt_ref[...] = (jnp.dot(g1, Wu2_ref[...], preferred_element_type=f32)
                     + bu2_ref[...]).T


def _tc_edge(S, T, edge_attr_t, p):
    f32 = jnp.float32
    Wkv = jnp.concatenate([p["WkN"], p["WvN"]], axis=1)
    We = jnp.concatenate([p["WkE"], p["WvE"]], axis=1)
    Wen, Wee = We[:D], We[D:]
    be = jnp.concatenate([p["bkE"], p["bvE"]])[None, :]
    Wb1n, Wb1e = p["Wb1"][:D], p["Wb1"][D:]
    Wu1a, Wu1b, Wu1c = p["Weu1"][:D], p["Weu1"][D:EF], p["Weu1"][EF:]
    sel = (jnp.arange(D)[:, None] // DH == jnp.arange(H)[None, :]).astype(f32)
    selt = sel.T
    row = lambda v: v[None, :]
    weights = [
        Wkv, p["Wq"], Wen, Wee, be,
        Wb1n, Wb1e, row(p["bb1"]), p["Wb2"], row(p["bb2"]),
        Wu1a, Wu1b, Wu1c, row(p["beu1"]), p["Weu2"], row(p["beu2"]),
        row(p["g_kv"]), row(p["b_kv"]), row(p["g_q"]), row(p["b_q"]),
        row(p["g_en"]), row(p["b_en"]),
        row(p["inv_temp"]), sel, selt,
    ]
    full = lambda a: pl.BlockSpec(a.shape, lambda i: (0,) * a.ndim)
    grid = E // BE
    return pl.pallas_call(
        _edge_body,
        grid=(grid,),
        in_specs=[
            pl.BlockSpec((BE, D), lambda i: (i, 0)),
            pl.BlockSpec((BE, D), lambda i: (i, 0)),
            pl.BlockSpec((ED, BE), lambda i: (0, i)),
        ] + [full(w) for w in weights],
        out_specs=[
            pl.BlockSpec((BE, D), lambda i: (i, 0)),
            pl.BlockSpec((H, BE), lambda i: (0, i)),
            pl.BlockSpec((ED, BE), lambda i: (0, i)),
        ],
        out_shape=[
            jax.ShapeDtypeStruct((E, D), f32),
            jax.ShapeDtypeStruct((H, E), f32),
            jax.ShapeDtypeStruct((ED, E), f32),
        ],
        compiler_params=pltpu.CompilerParams(
            dimension_semantics=("arbitrary",),
        ),
    )(S, T, edge_attr_t, *weights)


def _erf(x):
    a1, a2, a3, a4, a5 = (0.254829592, -0.284496736, 1.421413741,
                          -1.453152027, 1.061405429)
    sgn = jnp.sign(x)
    ax = jnp.abs(x)
    t = 1.0 / (1.0 + 0.3275911 * ax)
    poly = ((((a5 * t + a4) * t + a3) * t + a2) * t + a1) * t
    return sgn * (1.0 - poly * jnp.exp(-ax * ax))


def _node_body(Pn_ref, Pe_ref, tgt_ref,
               Wout_ref, bout_ref, Wg_ref, bg_ref, Wu_ref, bu_ref,
               Wd_ref, bd_ref, g1_ref, b1_ref, g2_ref, b2_ref,
               sc_ref, selt_ref, y_ref):
    f32 = jnp.float32
    numer = Pn_ref[...]
    s16 = Pe_ref[...]
    sb = jnp.dot(s16, selt_ref[...], preferred_element_type=f32)
    att = numer / (sb + 1e-16)
    out = jnp.dot(att, Wout_ref[...], preferred_element_type=f32) + bout_ref[...]
    res_scale = sc_ref[0, 0]
    ffn_scale = sc_ref[0, 1]
    y = tgt_ref[...] + res_scale * out
    y = _ln_rows(y) * g1_ref[...] + b1_ref[...]
    gate = jnp.dot(y, Wg_ref[...], preferred_element_type=f32) + bg_ref[...]
    up = jnp.dot(y, Wu_ref[...], preferred_element_type=f32) + bu_ref[...]
    gelu = up * 0.5 * (1.0 + _erf(up * 0.7071067811865475))
    yff = jnp.dot(gate * gelu, Wd_ref[...], preferred_element_type=f32) + bd_ref[...]
    y = y + ffn_scale * yff
    y_ref[...] = _ln_rows(y) * g2_ref[...] + b2_ref[...]


def _tc_node(Pn, Pe, target, p):
    f32 = jnp.float32
    selt16 = (jnp.arange(ED)[:, None] == jnp.arange(D)[None, :] // DH).astype(f32)
    row = lambda v: v[None, :]
    scales = jnp.concatenate([p["res_scale"], p["ffn_scale"]])[None, :]
    weights = [
        p["Wout"], row(p["bout"]), p["Wg"], row(p["bg"]), p["Wu"], row(p["bu"]),
        p["Wd"], row(p["bd"]), row(p["g_1"]), row(p["b_1"]),
        row(p["g_2"]), row(p["b_2"]), scales, selt16,
    ]
    full = lambda a: pl.BlockSpec(a.shape, lambda i: (0,) * a.ndim)
    grid = N // BN
    return pl.pallas_call(
        _node_body,
        grid=(grid,),
        in_specs=[
            pl.BlockSpec((BN, D), lambda i: (i, 0)),
            pl.BlockSpec((BN, ED), lambda i: (i, 0)),
            pl.BlockSpec((BN, D), lambda i: (i, 0)),
        ] + [full(w) for w in weights],
        out_specs=pl.BlockSpec((BN, D), lambda i: (i, 0)),
        out_shape=jax.ShapeDtypeStruct((N, D), f32),
        compiler_params=pltpu.CompilerParams(
            dimension_semantics=("arbitrary",),
        ),
    )(Pn, Pe, target, *weights)


def kernel(source, target, edge_index, edge_attr, params):
    sidx3 = edge_index[0].reshape(NW, NG, G)
    tidx3 = edge_index[1].reshape(NW, NG, G)
    S, T = _sc_gather(source, target, sidx3, tidx3)
    numer, ex_t, upd_t = _tc_edge(S, T, edge_attr.T, params)
    ex16 = jnp.concatenate(
        [ex_t.T, jnp.zeros((E, ED - H), jnp.float32)], axis=-1)
    zN = jnp.zeros((ACC_ROWS, D), jnp.float32)
    zE = jnp.zeros((ACC_ROWS, ED), jnp.float32)
    Pn = _make_sc_scatter(D, True)(numer, edge_index[1], zN)
    Pe = _make_sc_scatter(ED, False)(ex16, edge_index[1], zE)
    y = _tc_node(Pn, Pe, target, params)
    return (y, upd_t.T)


# speedup vs baseline: 1.1988x; 1.1988x over previous
import functools

import jax
import jax.numpy as jnp
from jax import lax
from jax.experimental import pallas as pl
from jax.experimental.pallas import tpu as pltpu
from jax.experimental.pallas import tpu_sc as plsc

N = 10000
E = 320000
D = 128
H = 4
DH = 32
ED = 16
EF = D + ED
HB = max(32, EF // 2)
FF = 4 * D
EUH = max(EF, D)

NC = 2
NS = 16
NW = NC * NS
CH = 5
EC = E // CH
G = 80
EPW = EC // NW
NG = EPW // G
EPT = EC // NS
NGS = EPT // G

HALF = N // NC
ACC_ROWS = HALF + 8

BE = 2560
BN = 2000


def _sc_gather(source, target, sidx3, tidx3):
    mesh = plsc.VectorSubcoreMesh(core_axis_name="c", subcore_axis_name="s")

    @functools.partial(
        pl.kernel,
        out_type=(
            jax.ShapeDtypeStruct((EC, D), jnp.float32),
            jax.ShapeDtypeStruct((EC, D), jnp.float32),
        ),
        mesh=mesh,
        scratch_types=[
            pltpu.VMEM((NG, G), jnp.int32),
            pltpu.VMEM((NG, G), jnp.int32),
            pltpu.VMEM((2, G, D), jnp.float32),
            pltpu.VMEM((2, G, D), jnp.float32),
            pltpu.SemaphoreType.DMA,
            pltpu.SemaphoreType.DMA,
            pltpu.SemaphoreType.DMA,
            pltpu.SemaphoreType.DMA,
            pltpu.SemaphoreType.DMA,
            pltpu.SemaphoreType.DMA,
            pltpu.SemaphoreType.DMA,
            pltpu.SemaphoreType.DMA,
        ],
    )
    def k(src_hbm, tgt_hbm, sidx_hbm, tidx_hbm, outS, outT,
          sidx_v, tidx_v, rowS, rowT, sS0, sS1, sT0, sT1, w0, w1, w2, w3):
        wid = lax.axis_index("s") * NC + lax.axis_index("c")
        base = wid * EPW
        pltpu.sync_copy(sidx_hbm.at[wid], sidx_v)
        pltpu.sync_copy(tidx_hbm.at[wid], tidx_v)

        def pair(j0, j1):
            gS0 = pltpu.async_copy(src_hbm.at[sidx_v.at[j0]], rowS.at[0], sS0)
            gT0 = pltpu.async_copy(tgt_hbm.at[tidx_v.at[j0]], rowT.at[0], sT0)
            gS1 = pltpu.async_copy(src_hbm.at[sidx_v.at[j1]], rowS.at[1], sS1)
            gT1 = pltpu.async_copy(tgt_hbm.at[tidx_v.at[j1]], rowT.at[1], sT1)
            gS0.wait()
            wS0 = pltpu.async_copy(rowS.at[0], outS.at[pl.ds(base + j0 * G, G)], w0)
            gT0.wait()
            wT0 = pltpu.async_copy(rowT.at[0], outT.at[pl.ds(base + j0 * G, G)], w1)
            gS1.wait()
            wS1 = pltpu.async_copy(rowS.at[1], outS.at[pl.ds(base + j1 * G, G)], w2)
            gT1.wait()
            wT1 = pltpu.async_copy(rowT.at[1], outT.at[pl.ds(base + j1 * G, G)], w3)
            wS0.wait()
            wT0.wait()
            wS1.wait()
            wT1.wait()

        def body(jj, carry):
            pair(jj * 2, jj * 2 + 1)
            return carry

        lax.fori_loop(0, NG // 2, body, 0)
        if NG % 2:
            j = NG - 1
            gS0 = pltpu.async_copy(src_hbm.at[sidx_v.at[j]], rowS.at[0], sS0)
            gT0 = pltpu.async_copy(tgt_hbm.at[tidx_v.at[j]], rowT.at[0], sT0)
            gS0.wait()
            pltpu.sync_copy(rowS.at[0], outS.at[pl.ds(base + j * G, G)])
            gT0.wait()
            pltpu.sync_copy(rowT.at[0], outT.at[pl.ds(base + j * G, G)])

    return k(source, target, sidx3, tidx3)


def _make_sc_scatter(W, tc_tiling):
    mesh = plsc.VectorSubcoreMesh(core_axis_name="c", subcore_axis_name="s")

    @functools.partial(
        pl.kernel,
        out_type=jax.ShapeDtypeStruct((N, W), jnp.float32),
        mesh=mesh,
        scratch_types=[
            pltpu.VMEM((2, G), jnp.int32),
            pltpu.VMEM((2, G), jnp.int32),
            pltpu.VMEM((2, G, W), jnp.float32),
            pltpu.VMEM_SHARED((ACC_ROWS, W), jnp.float32),
            pltpu.SemaphoreType.DMA,
            pltpu.SemaphoreType.DMA,
            pltpu.SemaphoreType.DMA,
            pltpu.SemaphoreType.DMA,
        ],
        compiler_params=pltpu.CompilerParams(use_tc_tiling_on_sc=tc_tiling),
    )
    def k(pay_hbm, tidx_hbm, zP_hbm, outP,
          idx_v, map_v, rowP, accP, sI0, sI1, sP0, sP1):
        cid = lax.axis_index("c")
        sid = lax.axis_index("s")
        tbase = sid * EPT
        nbase = cid * HALF

        @pl.when(sid == 0)
        def _init():
            pltpu.sync_copy(zP_hbm, accP)

        plsc.subcore_barrier()

        def chunk(j, slot, sI, sP):
            lI = pltpu.async_copy(tidx_hbm.at[pl.ds(tbase + j * G, G)],
                                  idx_v.at[slot], sI)
            lP = pltpu.async_copy(pay_hbm.at[pl.ds(tbase + j * G, G)],
                                  rowP.at[slot], sP)
            lI.wait()
            for kk in range(G // 16):
                v = idx_v[slot, pl.ds(kk * 16, 16)]
                loc = v - nbase
                inb = (loc >= 0) & (loc < HALF)
                map_v[slot, pl.ds(kk * 16, 16)] = jnp.where(inb, loc, HALF)
            lP.wait()
            pltpu.sync_copy(rowP.at[slot], accP.at[map_v.at[slot]], add=True)

        def body(jj, carry):
            chunk(jj * 2, 0, sI0, sP0)
            chunk(jj * 2 + 1, 1, sI1, sP1)
            return carry

        lax.fori_loop(0, NGS // 2, body, 0)
        plsc.subcore_barrier()

        @pl.when(sid == 0)
        def _dump():
            pltpu.sync_copy(accP.at[pl.ds(0, HALF)],
                            outP.at[pl.ds(cid * HALF, HALF)])

    return k


def _ln_rows(x, eps=1e-5):
    mu = jnp.mean(x, axis=-1, keepdims=True)
    v = jnp.mean(x * x, axis=-1, keepdims=True) - mu * mu
    return (x - mu) * lax.rsqrt(v + eps)


def _edge_body(S_ref, T_ref, EAT_ref,
               Wkv_ref, Wq_ref, Wen_ref, Wee_ref, be_ref,
               Wb1n_ref, Wb1e_ref, bb1_ref, Wb2_ref, bb2_ref,
               Wu1a_ref, Wu1b_ref, Wu1c_ref, bu1_ref, Wu2_ref, bu2_ref,
               gkv_ref, bkv_ref, gq_ref, bq_ref, gen_ref, ben_ref,
               it_ref, sel_ref, selt_ref,
               num_ref, ext_ref, updt_ref):
    f32 = jnp.float32
    s = S_ref[...]
    t = T_ref[...]
    ea = EAT_ref[...].T
    sn = _ln_rows(s) * gkv_ref[...] + bkv_ref[...]
    tn = _ln_rows(t) * gq_ref[...] + bq_ref[...]
    kv = jnp.dot(sn, Wkv_ref[...], preferred_element_type=f32)
    k_n = kv[:, :D]
    v_n = kv[:, D:]
    q = jnp.dot(tn, Wq_ref[...], preferred_element_type=f32)
    pw = s * t
    ssum = jnp.sum(pw, axis=-1, keepdims=True) + jnp.sum(ea, axis=-1, keepdims=True)
    ssq = jnp.sum(pw * pw, axis=-1, keepdims=True) + jnp.sum(ea * ea, axis=-1, keepdims=True)
    mu = ssum / EF
    var = ssq / EF - mu * mu
    inv = lax.rsqrt(var + 1e-5)
    efn = (pw - mu) * inv * gen_ref[:, :D] + ben_ref[:, :D]
    efe = (ea - mu) * inv * gen_ref[:, D:] + ben_ref[:, D:]
    kve = (jnp.dot(efn, Wen_ref[...], preferred_element_type=f32)
           + jnp.dot(efe, Wee_ref[...], preferred_element_type=f32)
           + be_ref[...])
    sk = k_n + kve[:, :D]
    sv = v_n + kve[:, D:]
    h = jax.nn.relu(jnp.dot(efn, Wb1n_ref[...], preferred_element_type=f32)
                    + jnp.dot(efe, Wb1e_ref[...], preferred_element_type=f32)
                    + bb1_ref[...])
    bias = jnp.dot(h, Wb2_ref[...], preferred_element_type=f32) + bb2_ref[...]
    prod = q * sk
    logits = (jnp.dot(prod, sel_ref[...], preferred_element_type=f32)
              * it_ref[...] + bias)
    ex = jnp.exp(logits)
    exb = jnp.dot(ex, selt_ref[...], preferred_element_type=f32)
    num_ref[...] = exb * sv
    ext_ref[...] = jnp.concatenate(
        [ex, jnp.zeros((ex.shape[0], ED - H), f32)], axis=-1).T
    g1 = jax.nn.relu(jnp.dot(efn, Wu1a_ref[...], preferred_element_type=f32)
                     + jnp.dot(efe, Wu1b_ref[...], preferred_element_type=f32)
                     + jnp.dot(sv, Wu1c_ref[...], preferred_element_type=f32)
                     + bu1_ref[...])
    updt_ref[...] = (jnp.dot(g1, Wu2_ref[...], preferred_element_type=f32)
                     + bu2_ref[...]).T


def _edge_weights(p):
    f32 = jnp.float32
    Wkv = jnp.concatenate([p["WkN"], p["WvN"]], axis=1)
    We = jnp.concatenate([p["WkE"], p["WvE"]], axis=1)
    Wen, Wee = We[:D], We[D:]
    be = jnp.concatenate([p["bkE"], p["bvE"]])[None, :]
    Wb1n, Wb1e = p["Wb1"][:D], p["Wb1"][D:]
    Wu1a, Wu1b, Wu1c = p["Weu1"][:D], p["Weu1"][D:EF], p["Weu1"][EF:]
    sel = (jnp.arange(D)[:, None] // DH == jnp.arange(H)[None, :]).astype(f32)
    selt = sel.T
    row = lambda v: v[None, :]
    return [
        Wkv, p["Wq"], Wen, Wee, be,
        Wb1n, Wb1e, row(p["bb1"]), p["Wb2"], row(p["bb2"]),
        Wu1a, Wu1b, Wu1c, row(p["beu1"]), p["Weu2"], row(p["beu2"]),
        row(p["g_kv"]), row(p["b_kv"]), row(p["g_q"]), row(p["b_q"]),
        row(p["g_en"]), row(p["b_en"]),
        row(p["inv_temp"]), sel, selt,
    ]


def _tc_edge(S, T, edge_attr_t, weights):
    f32 = jnp.float32
    full = lambda a: pl.BlockSpec(a.shape, lambda i: (0,) * a.ndim)
    grid = EC // BE
    return pl.pallas_call(
        _edge_body,
        grid=(grid,),
        in_specs=[
            pl.BlockSpec((BE, D), lambda i: (i, 0)),
            pl.BlockSpec((BE, D), lambda i: (i, 0)),
            pl.BlockSpec((ED, BE), lambda i: (0, i)),
        ] + [full(w) for w in weights],
        out_specs=[
            pl.BlockSpec((BE, D), lambda i: (i, 0)),
            pl.BlockSpec((ED, BE), lambda i: (0, i)),
            pl.BlockSpec((ED, BE), lambda i: (0, i)),
        ],
        out_shape=[
            jax.ShapeDtypeStruct((EC, D), f32),
            jax.ShapeDtypeStruct((ED, EC), f32),
            jax.ShapeDtypeStruct((ED, EC), f32),
        ],
        compiler_params=pltpu.CompilerParams(
            dimension_semantics=("arbitrary",),
        ),
    )(S, T, edge_attr_t, *weights)


def _erf(x):
    a1, a2, a3, a4, a5 = (0.254829592, -0.284496736, 1.421413741,
                          -1.453152027, 1.061405429)
    sgn = jnp.sign(x)
    ax = jnp.abs(x)
    t = 1.0 / (1.0 + 0.3275911 * ax)
    poly = ((((a5 * t + a4) * t + a3) * t + a2) * t + a1) * t
    return sgn * (1.0 - poly * jnp.exp(-ax * ax))


def _node_body(*refs):
    f32 = jnp.float32
    Pn_refs = refs[:CH]
    Pe_refs = refs[CH:2 * CH]
    (tgt_ref, Wout_ref, bout_ref, Wg_ref, bg_ref, Wu_ref, bu_ref,
     Wd_ref, bd_ref, g1_ref, b1_ref, g2_ref, b2_ref,
     sc_ref, selt_ref, y_ref) = refs[2 * CH:]
    numer = Pn_refs[0][...]
    s16 = Pe_refs[0][...]
    for r in Pn_refs[1:]:
        numer = numer + r[...]
    for r in Pe_refs[1:]:
        s16 = s16 + r[...]
    sb = jnp.dot(s16, selt_ref[...], preferred_element_type=f32)
    att = numer / (sb + 1e-16)
    out = jnp.dot(att, Wout_ref[...], preferred_element_type=f32) + bout_ref[...]
    res_scale = sc_ref[0, 0]
    ffn_scale = sc_ref[0, 1]
    y = tgt_ref[...] + res_scale * out
    y = _ln_rows(y) * g1_ref[...] + b1_ref[...]
    gate = jnp.dot(y, Wg_ref[...], preferred_element_type=f32) + bg_ref[...]
    up = jnp.dot(y, Wu_ref[...], preferred_element_type=f32) + bu_ref[...]
    gelu = up * 0.5 * (1.0 + _erf(up * 0.7071067811865475))
    yff = jnp.dot(gate * gelu, Wd_ref[...], preferred_element_type=f32) + bd_ref[...]
    y = y + ffn_scale * yff
    y_ref[...] = _ln_rows(y) * g2_ref[...] + b2_ref[...]


def _tc_node(Pns, Pes, target, p):
    f32 = jnp.float32
    selt16 = (jnp.arange(ED)[:, None] == jnp.arange(D)[None, :] // DH).astype(f32)
    row = lambda v: v[None, :]
    scales = jnp.concatenate([p["res_scale"], p["ffn_scale"]])[None, :]
    weights = [
        p["Wout"], row(p["bout"]), p["Wg"], row(p["bg"]), p["Wu"], row(p["bu"]),
        p["Wd"], row(p["bd"]), row(p["g_1"]), row(p["b_1"]),
        row(p["g_2"]), row(p["b_2"]), scales, selt16,
    ]
    full = lambda a: pl.BlockSpec(a.shape, lambda i: (0,) * a.ndim)
    grid = N // BN
    return pl.pallas_call(
        _node_body,
        grid=(grid,),
        in_specs=(
            [pl.BlockSpec((BN, D), lambda i: (i, 0)) for _ in Pns]
            + [pl.BlockSpec((BN, ED), lambda i: (i, 0)) for _ in Pes]
            + [pl.BlockSpec((BN, D), lambda i: (i, 0))]
            + [full(w) for w in weights]
        ),
        out_specs=pl.BlockSpec((BN, D), lambda i: (i, 0)),
        out_shape=jax.ShapeDtypeStruct((N, D), f32),
        compiler_params=pltpu.CompilerParams(
            dimension_semantics=("arbitrary",),
        ),
    )(*Pns, *Pes, target, *weights)


def kernel(source, target, edge_index, edge_attr, params):
    f32 = jnp.float32
    weights = _edge_weights(params)
    scatN = _make_sc_scatter(D, True)
    scatE = _make_sc_scatter(ED, False)
    zN = jnp.zeros((ACC_ROWS, D), f32)
    zE = jnp.zeros((ACC_ROWS, ED), f32)
    ea_t = edge_attr.T
    Pns, Pes, upds = [], [], []
    for c in range(CH):
        lo = c * EC
        sidx3 = lax.dynamic_slice_in_dim(edge_index[0], lo, EC).reshape(NW, NG, G)
        tidx3 = lax.dynamic_slice_in_dim(edge_index[1], lo, EC).reshape(NW, NG, G)
        tidx = lax.dynamic_slice_in_dim(edge_index[1], lo, EC)
        S, T = _sc_gather(source, target, sidx3, tidx3)
        numer, ex_t, upd_t = _tc_edge(
            S, T, lax.dynamic_slice_in_dim(ea_t, lo, EC, axis=1), weights)
        Pns.append(scatN(numer, tidx, zN))
        Pes.append(scatE(ex_t.T, tidx, zE))
        upds.append(upd_t)
    y = _tc_node(Pns, Pes, target, params)
    upd = jnp.concatenate(upds, axis=1).T
    return (y, upd)


# speedup vs baseline: 1.2555x; 1.0473x over previous
import functools

import jax
import jax.numpy as jnp
from jax import lax
from jax.experimental import pallas as pl
from jax.experimental.pallas import tpu as pltpu
from jax.experimental.pallas import tpu_sc as plsc

N = 10000
E = 320000
D = 128
H = 4
DH = 32
ED = 16
EF = D + ED
HB = max(32, EF // 2)
FF = 4 * D
EUH = max(EF, D)

NC = 2
NS = 16
NW = NC * NS
CH = 5
EC = E // CH
G = 80
EPW = EC // NW
NG = EPW // G
EPT = EC // NS
NGS = EPT // G

HALF = N // NC
ACC_ROWS = HALF + 8

BE = 2560
BN = 2000


def _sc_gather(source, target, sidx3, tidx3):
    mesh = plsc.VectorSubcoreMesh(core_axis_name="c", subcore_axis_name="s")

    @functools.partial(
        pl.kernel,
        out_type=(
            jax.ShapeDtypeStruct((EC, D), jnp.float32),
            jax.ShapeDtypeStruct((EC, D), jnp.float32),
        ),
        mesh=mesh,
        scratch_types=[
            pltpu.VMEM((NG, G), jnp.int32),
            pltpu.VMEM((NG, G), jnp.int32),
            pltpu.VMEM((2, G, D), jnp.float32),
            pltpu.VMEM((2, G, D), jnp.float32),
            pltpu.SemaphoreType.DMA,
            pltpu.SemaphoreType.DMA,
            pltpu.SemaphoreType.DMA,
            pltpu.SemaphoreType.DMA,
            pltpu.SemaphoreType.DMA,
            pltpu.SemaphoreType.DMA,
            pltpu.SemaphoreType.DMA,
            pltpu.SemaphoreType.DMA,
        ],
    )
    def k(src_hbm, tgt_hbm, sidx_hbm, tidx_hbm, outS, outT,
          sidx_v, tidx_v, rowS, rowT, sS0, sS1, sT0, sT1, w0, w1, w2, w3):
        wid = lax.axis_index("s") * NC + lax.axis_index("c")
        base = wid * EPW
        pltpu.sync_copy(sidx_hbm.at[wid], sidx_v)
        pltpu.sync_copy(tidx_hbm.at[wid], tidx_v)

        def pair(j0, j1):
            gS0 = pltpu.async_copy(src_hbm.at[sidx_v.at[j0]], rowS.at[0], sS0)
            gT0 = pltpu.async_copy(tgt_hbm.at[tidx_v.at[j0]], rowT.at[0], sT0)
            gS1 = pltpu.async_copy(src_hbm.at[sidx_v.at[j1]], rowS.at[1], sS1)
            gT1 = pltpu.async_copy(tgt_hbm.at[tidx_v.at[j1]], rowT.at[1], sT1)
            gS0.wait()
            wS0 = pltpu.async_copy(rowS.at[0], outS.at[pl.ds(base + j0 * G, G)], w0)
            gT0.wait()
            wT0 = pltpu.async_copy(rowT.at[0], outT.at[pl.ds(base + j0 * G, G)], w1)
            gS1.wait()
            wS1 = pltpu.async_copy(rowS.at[1], outS.at[pl.ds(base + j1 * G, G)], w2)
            gT1.wait()
            wT1 = pltpu.async_copy(rowT.at[1], outT.at[pl.ds(base + j1 * G, G)], w3)
            wS0.wait()
            wT0.wait()
            wS1.wait()
            wT1.wait()

        def body(jj, carry):
            pair(jj * 2, jj * 2 + 1)
            return carry

        lax.fori_loop(0, NG // 2, body, 0)
        if NG % 2:
            j = NG - 1
            gS0 = pltpu.async_copy(src_hbm.at[sidx_v.at[j]], rowS.at[0], sS0)
            gT0 = pltpu.async_copy(tgt_hbm.at[tidx_v.at[j]], rowT.at[0], sT0)
            gS0.wait()
            pltpu.sync_copy(rowS.at[0], outS.at[pl.ds(base + j * G, G)])
            gT0.wait()
            pltpu.sync_copy(rowT.at[0], outT.at[pl.ds(base + j * G, G)])

    return k(source, target, sidx3, tidx3)


NSL = 4


def _make_sc_scatter(W, tc_tiling):
    mesh = plsc.VectorSubcoreMesh(core_axis_name="c", subcore_axis_name="s")

    @functools.partial(
        pl.kernel,
        out_type=jax.ShapeDtypeStruct((N, W), jnp.float32),
        mesh=mesh,
        scratch_types=[
            pltpu.VMEM((NSL, G), jnp.int32),
            pltpu.VMEM((NSL, G), jnp.int32),
            pltpu.VMEM((NSL, G, W), jnp.float32),
            pltpu.VMEM_SHARED((ACC_ROWS, W), jnp.float32),
            [pltpu.SemaphoreType.DMA] * NSL,
            [pltpu.SemaphoreType.DMA] * NSL,
            [pltpu.SemaphoreType.DMA] * NSL,
        ],
        compiler_params=pltpu.CompilerParams(use_tc_tiling_on_sc=tc_tiling),
    )
    def k(pay_hbm, tidx_hbm, prev_hbm, zrow_hbm, tok_hbm, outP,
          idx_v, map_v, rowP, accP, sI, sP, sS):
        cid = lax.axis_index("c")
        sid = lax.axis_index("s")
        tbase = sid * EPT
        nbase = cid * HALF

        @pl.when(sid == 0)
        def _init():
            pltpu.sync_copy(prev_hbm.at[pl.ds(cid * HALF, HALF)],
                            accP.at[pl.ds(0, HALF)])
            pltpu.sync_copy(zrow_hbm, accP.at[pl.ds(HALF, 8)])

        plsc.subcore_barrier()

        def load(j, slot):
            lI = pltpu.async_copy(tidx_hbm.at[pl.ds(tbase + j * G, G)],
                                  idx_v.at[slot], sI[slot])
            lP = pltpu.async_copy(pay_hbm.at[pl.ds(tbase + j * G, G)],
                                  rowP.at[slot], sP[slot])
            return lI, lP

        def scat(j, slot, lI, lP):
            lI.wait()
            for kk in range(G // 16):
                v = idx_v[slot, pl.ds(kk * 16, 16)]
                loc = v - nbase
                inb = (loc >= 0) & (loc < HALF)
                map_v[slot, pl.ds(kk * 16, 16)] = jnp.where(inb, loc, HALF)
            lP.wait()
            return pltpu.async_copy(rowP.at[slot], accP.at[map_v.at[slot]],
                                    sS[slot], add=True)

        def group(j0, nsl):
            ls = [load(j0 + u, u) for u in range(nsl)]
            ws = [scat(j0 + u, u, *ls[u]) for u in range(nsl)]
            for w in ws:
                w.wait()

        def body(jj, carry):
            group(jj * NSL, NSL)
            return carry

        lax.fori_loop(0, NGS // NSL, body, 0)
        if NGS % NSL:
            group(NGS - NGS % NSL, NGS % NSL)
        plsc.subcore_barrier()

        @pl.when(sid == 0)
        def _dump():
            pltpu.sync_copy(accP.at[pl.ds(0, HALF)],
                            outP.at[pl.ds(cid * HALF, HALF)])

    return k


def _ln_rows(x, eps=1e-5):
    mu = jnp.mean(x, axis=-1, keepdims=True)
    v = jnp.mean(x * x, axis=-1, keepdims=True) - mu * mu
    return (x - mu) * lax.rsqrt(v + eps)


def _edge_body(S_ref, T_ref, EAT_ref,
               Wkv_ref, Wq_ref, Wen_ref, Wee_ref, be_ref,
               Wb1n_ref, Wb1e_ref, bb1_ref, Wb2_ref, bb2_ref,
               Wu1a_ref, Wu1b_ref, Wu1c_ref, bu1_ref, Wu2_ref, bu2_ref,
               gkv_ref, bkv_ref, gq_ref, bq_ref, gen_ref, ben_ref,
               it_ref, sel_ref, selt_ref,
               num_ref, ext_ref, updt_ref):
    f32 = jnp.float32
    s = S_ref[...]
    t = T_ref[...]
    ea = EAT_ref[...].T
    sn = _ln_rows(s) * gkv_ref[...] + bkv_ref[...]
    tn = _ln_rows(t) * gq_ref[...] + bq_ref[...]
    kv = jnp.dot(sn, Wkv_ref[...], preferred_element_type=f32)
    k_n = kv[:, :D]
    v_n = kv[:, D:]
    q = jnp.dot(tn, Wq_ref[...], preferred_element_type=f32)
    pw = s * t
    ssum = jnp.sum(pw, axis=-1, keepdims=True) + jnp.sum(ea, axis=-1, keepdims=True)
    ssq = jnp.sum(pw * pw, axis=-1, keepdims=True) + jnp.sum(ea * ea, axis=-1, keepdims=True)
    mu = ssum / EF
    var = ssq / EF - mu * mu
    inv = lax.rsqrt(var + 1e-5)
    efn = (pw - mu) * inv * gen_ref[:, :D] + ben_ref[:, :D]
    efe = (ea - mu) * inv * gen_ref[:, D:] + ben_ref[:, D:]
    kve = (jnp.dot(efn, Wen_ref[...], preferred_element_type=f32)
           + jnp.dot(efe, Wee_ref[...], preferred_element_type=f32)
           + be_ref[...])
    sk = k_n + kve[:, :D]
    sv = v_n + kve[:, D:]
    h = jax.nn.relu(jnp.dot(efn, Wb1n_ref[...], preferred_element_type=f32)
                    + jnp.dot(efe, Wb1e_ref[...], preferred_element_type=f32)
                    + bb1_ref[...])
    bias = jnp.dot(h, Wb2_ref[...], preferred_element_type=f32) + bb2_ref[...]
    prod = q * sk
    logits = (jnp.dot(prod, sel_ref[...], preferred_element_type=f32)
              * it_ref[...] + bias)
    ex = jnp.exp(logits)
    exb = jnp.dot(ex, selt_ref[...], preferred_element_type=f32)
    num_ref[...] = exb * sv
    ext_ref[...] = jnp.concatenate(
        [ex, jnp.zeros((ex.shape[0], ED - H), f32)], axis=-1).T
    g1 = jax.nn.relu(jnp.dot(efn, Wu1a_ref[...], preferred_element_type=f32)
                     + jnp.dot(efe, Wu1b_ref[...], preferred_element_type=f32)
                     + jnp.dot(sv, Wu1c_ref[...], preferred_element_type=f32)
                     + bu1_ref[...])
    updt_ref[...] = (jnp.dot(g1, Wu2_ref[...], preferred_element_type=f32)
                     + bu2_ref[...]).T


def _edge_weights(p):
    f32 = jnp.float32
    Wkv = jnp.concatenate([p["WkN"], p["WvN"]], axis=1)
    We = jnp.concatenate([p["WkE"], p["WvE"]], axis=1)
    Wen, Wee = We[:D], We[D:]
    be = jnp.concatenate([p["bkE"], p["bvE"]])[None, :]
    Wb1n, Wb1e = p["Wb1"][:D], p["Wb1"][D:]
    Wu1a, Wu1b, Wu1c = p["Weu1"][:D], p["Weu1"][D:EF], p["Weu1"][EF:]
    sel = (jnp.arange(D)[:, None] // DH == jnp.arange(H)[None, :]).astype(f32)
    selt = sel.T
    row = lambda v: v[None, :]
    return [
        Wkv, p["Wq"], Wen, Wee, be,
        Wb1n, Wb1e, row(p["bb1"]), p["Wb2"], row(p["bb2"]),
        Wu1a, Wu1b, Wu1c, row(p["beu1"]), p["Weu2"], row(p["beu2"]),
        row(p["g_kv"]), row(p["b_kv"]), row(p["g_q"]), row(p["b_q"]),
        row(p["g_en"]), row(p["b_en"]),
        row(p["inv_temp"]), sel, selt,
    ]


def _tc_edge(S, T, edge_attr_t, weights):
    f32 = jnp.float32
    full = lambda a: pl.BlockSpec(a.shape, lambda i: (0,) * a.ndim)
    grid = EC // BE
    return pl.pallas_call(
        _edge_body,
        grid=(grid,),
        in_specs=[
            pl.BlockSpec((BE, D), lambda i: (i, 0)),
            pl.BlockSpec((BE, D), lambda i: (i, 0)),
            pl.BlockSpec((ED, BE), lambda i: (0, i)),
        ] + [full(w) for w in weights],
        out_specs=[
            pl.BlockSpec((BE, D), lambda i: (i, 0)),
            pl.BlockSpec((ED, BE), lambda i: (0, i)),
            pl.BlockSpec((ED, BE), lambda i: (0, i)),
        ],
        out_shape=[
            jax.ShapeDtypeStruct((EC, D), f32),
            jax.ShapeDtypeStruct((ED, EC), f32),
            jax.ShapeDtypeStruct((ED, EC), f32),
        ],
        compiler_params=pltpu.CompilerParams(
            dimension_semantics=("arbitrary",),
        ),
    )(S, T, edge_attr_t, *weights)


def _erf(x):
    a1, a2, a3, a4, a5 = (0.254829592, -0.284496736, 1.421413741,
                          -1.453152027, 1.061405429)
    sgn = jnp.sign(x)
    ax = jnp.abs(x)
    t = 1.0 / (1.0 + 0.3275911 * ax)
    poly = ((((a5 * t + a4) * t + a3) * t + a2) * t + a1) * t
    return sgn * (1.0 - poly * jnp.exp(-ax * ax))


def _node_body(Pn_ref, Pe_ref, tgt_ref,
               Wout_ref, bout_ref, Wg_ref, bg_ref, Wu_ref, bu_ref,
               Wd_ref, bd_ref, g1_ref, b1_ref, g2_ref, b2_ref,
               sc_ref, selt_ref, y_ref):
    f32 = jnp.float32
    numer = Pn_ref[...]
    s16 = Pe_ref[...]
    sb = jnp.dot(s16, selt_ref[...], preferred_element_type=f32)
    att = numer / (sb + 1e-16)
    out = jnp.dot(att, Wout_ref[...], preferred_element_type=f32) + bout_ref[...]
    res_scale = sc_ref[0, 0]
    ffn_scale = sc_ref[0, 1]
    y = tgt_ref[...] + res_scale * out
    y = _ln_rows(y) * g1_ref[...] + b1_ref[...]
    gate = jnp.dot(y, Wg_ref[...], preferred_element_type=f32) + bg_ref[...]
    up = jnp.dot(y, Wu_ref[...], preferred_element_type=f32) + bu_ref[...]
    gelu = up * 0.5 * (1.0 + _erf(up * 0.7071067811865475))
    yff = jnp.dot(gate * gelu, Wd_ref[...], preferred_element_type=f32) + bd_ref[...]
    y = y + ffn_scale * yff
    y_ref[...] = _ln_rows(y) * g2_ref[...] + b2_ref[...]


def _tc_node(Pn, Pe, target, p):
    f32 = jnp.float32
    selt16 = (jnp.arange(ED)[:, None] == jnp.arange(D)[None, :] // DH).astype(f32)
    row = lambda v: v[None, :]
    scales = jnp.concatenate([p["res_scale"], p["ffn_scale"]])[None, :]
    weights = [
        p["Wout"], row(p["bout"]), p["Wg"], row(p["bg"]), p["Wu"], row(p["bu"]),
        p["Wd"], row(p["bd"]), row(p["g_1"]), row(p["b_1"]),
        row(p["g_2"]), row(p["b_2"]), scales, selt16,
    ]
    full = lambda a: pl.BlockSpec(a.shape, lambda i: (0,) * a.ndim)
    grid = N // BN
    return pl.pallas_call(
        _node_body,
        grid=(grid,),
        in_specs=(
            [pl.BlockSpec((BN, D), lambda i: (i, 0)),
             pl.BlockSpec((BN, ED), lambda i: (i, 0)),
             pl.BlockSpec((BN, D), lambda i: (i, 0))]
            + [full(w) for w in weights]
        ),
        out_specs=pl.BlockSpec((BN, D), lambda i: (i, 0)),
        out_shape=jax.ShapeDtypeStruct((N, D), f32),
        compiler_params=pltpu.CompilerParams(
            dimension_semantics=("arbitrary",),
        ),
    )(Pn, Pe, target, *weights)


def kernel(source, target, edge_index, edge_attr, params):
    f32 = jnp.float32
    weights = _edge_weights(params)
    scatN = _make_sc_scatter(D, True)
    scatE = _make_sc_scatter(ED, False)
    zrowN = jnp.zeros((8, D), f32)
    zrowE = jnp.zeros((8, ED), f32)
    Pn = jnp.zeros((N, D), f32)
    Pe = jnp.zeros((N, ED), f32)
    ea_t = edge_attr.T
    upds = []
    for c in range(CH):
        lo = c * EC
        sidx3 = lax.dynamic_slice_in_dim(edge_index[0], lo, EC).reshape(NW, NG, G)
        tidx3 = lax.dynamic_slice_in_dim(edge_index[1], lo, EC).reshape(NW, NG, G)
        tidx = lax.dynamic_slice_in_dim(edge_index[1], lo, EC)
        S, T = _sc_gather(source, target, sidx3, tidx3)
        numer, ex_t, upd_t = _tc_edge(
            S, T, lax.dynamic_slice_in_dim(ea_t, lo, EC, axis=1), weights)
        Pn_new = scatN(numer, tidx, Pn, zrowN, Pe)
        Pe = scatE(ex_t.T, tidx, Pe, zrowE, Pn_new)
        Pn = Pn_new
        upds.append(upd_t)
    y = _tc_node(Pn, Pe, target, params)
    upd = jnp.concatenate(upds, axis=1).T
    return (y, upd)


# speedup vs baseline: 1.3008x; 1.0361x over previous
import functools

import jax
import jax.numpy as jnp
from jax import lax
from jax.experimental import pallas as pl
from jax.experimental.pallas import tpu as pltpu
from jax.experimental.pallas import tpu_sc as plsc

N = 10000
E = 320000
D = 128
H = 4
DH = 32
ED = 16
EF = D + ED
HB = max(32, EF // 2)
FF = 4 * D
EUH = max(EF, D)

NC = 2
NS = 16
NW = NC * NS
CH = 5
EC = E // CH
G = 80
EPW = EC // NW
NG = EPW // G
EPT = EC // NS
NGS = EPT // G

HALF = N // NC
ACC_ROWS = HALF + 8

BE = 2560
BN = 2000


def _sc_gather(source, target, sidx3, tidx3, tok):
    mesh = plsc.VectorSubcoreMesh(core_axis_name="c", subcore_axis_name="s")

    @functools.partial(
        pl.kernel,
        out_type=(
            jax.ShapeDtypeStruct((EC, D), jnp.float32),
            jax.ShapeDtypeStruct((EC, D), jnp.float32),
        ),
        mesh=mesh,
        scratch_types=[
            pltpu.VMEM((NG, G), jnp.int32),
            pltpu.VMEM((NG, G), jnp.int32),
            pltpu.VMEM((2, G, D), jnp.float32),
            pltpu.VMEM((2, G, D), jnp.float32),
            pltpu.SemaphoreType.DMA,
            pltpu.SemaphoreType.DMA,
            pltpu.SemaphoreType.DMA,
            pltpu.SemaphoreType.DMA,
            pltpu.SemaphoreType.DMA,
            pltpu.SemaphoreType.DMA,
            pltpu.SemaphoreType.DMA,
            pltpu.SemaphoreType.DMA,
        ],
    )
    def k(src_hbm, tgt_hbm, sidx_hbm, tidx_hbm, tok_hbm, outS, outT,
          sidx_v, tidx_v, rowS, rowT, sS0, sS1, sT0, sT1, w0, w1, w2, w3):
        wid = lax.axis_index("s") * NC + lax.axis_index("c")
        base = wid * EPW
        pltpu.sync_copy(sidx_hbm.at[wid], sidx_v)
        pltpu.sync_copy(tidx_hbm.at[wid], tidx_v)

        def pair(j0, j1):
            gS0 = pltpu.async_copy(src_hbm.at[sidx_v.at[j0]], rowS.at[0], sS0)
            gT0 = pltpu.async_copy(tgt_hbm.at[tidx_v.at[j0]], rowT.at[0], sT0)
            gS1 = pltpu.async_copy(src_hbm.at[sidx_v.at[j1]], rowS.at[1], sS1)
            gT1 = pltpu.async_copy(tgt_hbm.at[tidx_v.at[j1]], rowT.at[1], sT1)
            gS0.wait()
            wS0 = pltpu.async_copy(rowS.at[0], outS.at[pl.ds(base + j0 * G, G)], w0)
            gT0.wait()
            wT0 = pltpu.async_copy(rowT.at[0], outT.at[pl.ds(base + j0 * G, G)], w1)
            gS1.wait()
            wS1 = pltpu.async_copy(rowS.at[1], outS.at[pl.ds(base + j1 * G, G)], w2)
            gT1.wait()
            wT1 = pltpu.async_copy(rowT.at[1], outT.at[pl.ds(base + j1 * G, G)], w3)
            wS0.wait()
            wT0.wait()
            wS1.wait()
            wT1.wait()

        def body(jj, carry):
            pair(jj * 2, jj * 2 + 1)
            return carry

        lax.fori_loop(0, NG // 2, body, 0)
        if NG % 2:
            j = NG - 1
            gS0 = pltpu.async_copy(src_hbm.at[sidx_v.at[j]], rowS.at[0], sS0)
            gT0 = pltpu.async_copy(tgt_hbm.at[tidx_v.at[j]], rowT.at[0], sT0)
            gS0.wait()
            pltpu.sync_copy(rowS.at[0], outS.at[pl.ds(base + j * G, G)])
            gT0.wait()
            pltpu.sync_copy(rowT.at[0], outT.at[pl.ds(base + j * G, G)])

    return k(source, target, sidx3, tidx3, tok)


NSL = 4


def _make_sc_scatter(W, tc_tiling):
    mesh = plsc.VectorSubcoreMesh(core_axis_name="c", subcore_axis_name="s")

    @functools.partial(
        pl.kernel,
        out_type=jax.ShapeDtypeStruct((N, W), jnp.float32),
        mesh=mesh,
        scratch_types=[
            pltpu.VMEM((NSL, G), jnp.int32),
            pltpu.VMEM((NSL, G), jnp.int32),
            pltpu.VMEM((NSL, G, W), jnp.float32),
            pltpu.VMEM_SHARED((ACC_ROWS, W), jnp.float32),
            [pltpu.SemaphoreType.DMA] * NSL,
            [pltpu.SemaphoreType.DMA] * NSL,
            [pltpu.SemaphoreType.DMA] * NSL,
        ],
        compiler_params=pltpu.CompilerParams(use_tc_tiling_on_sc=tc_tiling),
    )
    def k(pay_hbm, tidx_hbm, prev_hbm, zrow_hbm, tok_hbm, outP,
          idx_v, map_v, rowP, accP, sI, sP, sS):
        cid = lax.axis_index("c")
        sid = lax.axis_index("s")
        tbase = sid * EPT
        nbase = cid * HALF

        @pl.when(sid == 0)
        def _init():
            pltpu.sync_copy(prev_hbm.at[pl.ds(cid * HALF, HALF)],
                            accP.at[pl.ds(0, HALF)])
            pltpu.sync_copy(zrow_hbm, accP.at[pl.ds(HALF, 8)])

        plsc.subcore_barrier()

        def load(j, slot):
            lI = pltpu.async_copy(tidx_hbm.at[pl.ds(tbase + j * G, G)],
                                  idx_v.at[slot], sI[slot])
            lP = pltpu.async_copy(pay_hbm.at[pl.ds(tbase + j * G, G)],
                                  rowP.at[slot], sP[slot])
            return lI, lP

        def scat(j, slot, lI, lP):
            lI.wait()
            for kk in range(G // 16):
                v = idx_v[slot, pl.ds(kk * 16, 16)]
                loc = v - nbase
                inb = (loc >= 0) & (loc < HALF)
                map_v[slot, pl.ds(kk * 16, 16)] = jnp.where(inb, loc, HALF)
            lP.wait()
            return pltpu.async_copy(rowP.at[slot], accP.at[map_v.at[slot]],
                                    sS[slot], add=True)

        def group(j0, nsl):
            ls = [load(j0 + u, u) for u in range(nsl)]
            ws = [scat(j0 + u, u, *ls[u]) for u in range(nsl)]
            for w in ws:
                w.wait()

        def body(jj, carry):
            group(jj * NSL, NSL)
            return carry

        lax.fori_loop(0, NGS // NSL, body, 0)
        if NGS % NSL:
            group(NGS - NGS % NSL, NGS % NSL)
        plsc.subcore_barrier()

        @pl.when(sid == 0)
        def _dump():
            pltpu.sync_copy(accP.at[pl.ds(0, HALF)],
                            outP.at[pl.ds(cid * HALF, HALF)])

    return k


def _ln_rows(x, eps=1e-5):
    mu = jnp.mean(x, axis=-1, keepdims=True)
    v = jnp.mean(x * x, axis=-1, keepdims=True) - mu * mu
    return (x - mu) * lax.rsqrt(v + eps)


def _edge_body(S_ref, T_ref, EAT_ref,
               Wkv_ref, Wq_ref, Wen_ref, Wee_ref, be_ref,
               Wb1n_ref, Wb1e_ref, bb1_ref, Wb2_ref, bb2_ref,
               Wu1a_ref, Wu1b_ref, Wu1c_ref, bu1_ref, Wu2_ref, bu2_ref,
               gkv_ref, bkv_ref, gq_ref, bq_ref, gen_ref, ben_ref,
               it_ref, sel_ref, selt_ref,
               num_ref, ext_ref, updt_ref):
    f32 = jnp.float32
    s = S_ref[...]
    t = T_ref[...]
    ea = EAT_ref[...].T
    sn = _ln_rows(s) * gkv_ref[...] + bkv_ref[...]
    tn = _ln_rows(t) * gq_ref[...] + bq_ref[...]
    kv = jnp.dot(sn, Wkv_ref[...], preferred_element_type=f32)
    k_n = kv[:, :D]
    v_n = kv[:, D:]
    q = jnp.dot(tn, Wq_ref[...], preferred_element_type=f32)
    pw = s * t
    ssum = jnp.sum(pw, axis=-1, keepdims=True) + jnp.sum(ea, axis=-1, keepdims=True)
    ssq = jnp.sum(pw * pw, axis=-1, keepdims=True) + jnp.sum(ea * ea, axis=-1, keepdims=True)
    mu = ssum / EF
    var = ssq / EF - mu * mu
    inv = lax.rsqrt(var + 1e-5)
    efn = (pw - mu) * inv * gen_ref[:, :D] + ben_ref[:, :D]
    efe = (ea - mu) * inv * gen_ref[:, D:] + ben_ref[:, D:]
    kve = (jnp.dot(efn, Wen_ref[...], preferred_element_type=f32)
           + jnp.dot(efe, Wee_ref[...], preferred_element_type=f32)
           + be_ref[...])
    sk = k_n + kve[:, :D]
    sv = v_n + kve[:, D:]
    h = jax.nn.relu(jnp.dot(efn, Wb1n_ref[...], preferred_element_type=f32)
                    + jnp.dot(efe, Wb1e_ref[...], preferred_element_type=f32)
                    + bb1_ref[...])
    bias = jnp.dot(h, Wb2_ref[...], preferred_element_type=f32) + bb2_ref[...]
    prod = q * sk
    logits = (jnp.dot(prod, sel_ref[...], preferred_element_type=f32)
              * it_ref[...] + bias)
    ex = jnp.exp(logits)
    exb = jnp.dot(ex, selt_ref[...], preferred_element_type=f32)
    num_ref[...] = exb * sv
    ext_ref[...] = jnp.concatenate(
        [ex, jnp.zeros((ex.shape[0], ED - H), f32)], axis=-1).T
    g1 = jax.nn.relu(jnp.dot(efn, Wu1a_ref[...], preferred_element_type=f32)
                     + jnp.dot(efe, Wu1b_ref[...], preferred_element_type=f32)
                     + jnp.dot(sv, Wu1c_ref[...], preferred_element_type=f32)
                     + bu1_ref[...])
    updt_ref[...] = (jnp.dot(g1, Wu2_ref[...], preferred_element_type=f32)
                     + bu2_ref[...]).T


def _edge_weights(p):
    f32 = jnp.float32
    Wkv = jnp.concatenate([p["WkN"], p["WvN"]], axis=1)
    We = jnp.concatenate([p["WkE"], p["WvE"]], axis=1)
    Wen, Wee = We[:D], We[D:]
    be = jnp.concatenate([p["bkE"], p["bvE"]])[None, :]
    Wb1n, Wb1e = p["Wb1"][:D], p["Wb1"][D:]
    Wu1a, Wu1b, Wu1c = p["Weu1"][:D], p["Weu1"][D:EF], p["Weu1"][EF:]
    sel = (jnp.arange(D)[:, None] // DH == jnp.arange(H)[None, :]).astype(f32)
    selt = sel.T
    row = lambda v: v[None, :]
    return [
        Wkv, p["Wq"], Wen, Wee, be,
        Wb1n, Wb1e, row(p["bb1"]), p["Wb2"], row(p["bb2"]),
        Wu1a, Wu1b, Wu1c, row(p["beu1"]), p["Weu2"], row(p["beu2"]),
        row(p["g_kv"]), row(p["b_kv"]), row(p["g_q"]), row(p["b_q"]),
        row(p["g_en"]), row(p["b_en"]),
        row(p["inv_temp"]), sel, selt,
    ]


def _tc_edge(S, T, edge_attr_t, weights):
    f32 = jnp.float32
    full = lambda a: pl.BlockSpec(a.shape, lambda i: (0,) * a.ndim)
    grid = EC // BE
    return pl.pallas_call(
        _edge_body,
        grid=(grid,),
        in_specs=[
            pl.BlockSpec((BE, D), lambda i: (i, 0)),
            pl.BlockSpec((BE, D), lambda i: (i, 0)),
            pl.BlockSpec((ED, BE), lambda i: (0, i)),
        ] + [full(w) for w in weights],
        out_specs=[
            pl.BlockSpec((BE, D), lambda i: (i, 0)),
            pl.BlockSpec((ED, BE), lambda i: (0, i)),
            pl.BlockSpec((ED, BE), lambda i: (0, i)),
        ],
        out_shape=[
            jax.ShapeDtypeStruct((EC, D), f32),
            jax.ShapeDtypeStruct((ED, EC), f32),
            jax.ShapeDtypeStruct((ED, EC), f32),
        ],
        compiler_params=pltpu.CompilerParams(
            dimension_semantics=("arbitrary",),
        ),
    )(S, T, edge_attr_t, *weights)


def _erf(x):
    a1, a2, a3, a4, a5 = (0.254829592, -0.284496736, 1.421413741,
                          -1.453152027, 1.061405429)
    sgn = jnp.sign(x)
    ax = jnp.abs(x)
    t = 1.0 / (1.0 + 0.3275911 * ax)
    poly = ((((a5 * t + a4) * t + a3) * t + a2) * t + a1) * t
    return sgn * (1.0 - poly * jnp.exp(-ax * ax))


def _node_body(Pn_ref, Pe_ref, tgt_ref,
               Wout_ref, bout_ref, Wg_ref, bg_ref, Wu_ref, bu_ref,
               Wd_ref, bd_ref, g1_ref, b1_ref, g2_ref, b2_ref,
               sc_ref, selt_ref, y_ref):
    f32 = jnp.float32
    numer = Pn_ref[...]
    s16 = Pe_ref[...]
    sb = jnp.dot(s16, selt_ref[...], preferred_element_type=f32)
    att = numer / (sb + 1e-16)
    out = jnp.dot(att, Wout_ref[...], preferred_element_type=f32) + bout_ref[...]
    res_scale = sc_ref[0, 0]
    ffn_scale = sc_ref[0, 1]
    y = tgt_ref[...] + res_scale * out
    y = _ln_rows(y) * g1_ref[...] + b1_ref[...]
    gate = jnp.dot(y, Wg_ref[...], preferred_element_type=f32) + bg_ref[...]
    up = jnp.dot(y, Wu_ref[...], preferred_element_type=f32) + bu_ref[...]
    gelu = up * 0.5 * (1.0 + _erf(up * 0.7071067811865475))
    yff = jnp.dot(gate * gelu, Wd_ref[...], preferred_element_type=f32) + bd_ref[...]
    y = y + ffn_scale * yff
    y_ref[...] = _ln_rows(y) * g2_ref[...] + b2_ref[...]


def _tc_node(Pn, Pe, target, p):
    f32 = jnp.float32
    selt16 = (jnp.arange(ED)[:, None] == jnp.arange(D)[None, :] // DH).astype(f32)
    row = lambda v: v[None, :]
    scales = jnp.concatenate([p["res_scale"], p["ffn_scale"]])[None, :]
    weights = [
        p["Wout"], row(p["bout"]), p["Wg"], row(p["bg"]), p["Wu"], row(p["bu"]),
        p["Wd"], row(p["bd"]), row(p["g_1"]), row(p["b_1"]),
        row(p["g_2"]), row(p["b_2"]), scales, selt16,
    ]
    full = lambda a: pl.BlockSpec(a.shape, lambda i: (0,) * a.ndim)
    grid = N // BN
    return pl.pallas_call(
        _node_body,
        grid=(grid,),
        in_specs=(
            [pl.BlockSpec((BN, D), lambda i: (i, 0)),
             pl.BlockSpec((BN, ED), lambda i: (i, 0)),
             pl.BlockSpec((BN, D), lambda i: (i, 0))]
            + [full(w) for w in weights]
        ),
        out_specs=pl.BlockSpec((BN, D), lambda i: (i, 0)),
        out_shape=jax.ShapeDtypeStruct((N, D), f32),
        compiler_params=pltpu.CompilerParams(
            dimension_semantics=("arbitrary",),
        ),
    )(Pn, Pe, target, *weights)


def kernel(source, target, edge_index, edge_attr, params):
    f32 = jnp.float32
    weights = _edge_weights(params)
    scatN = _make_sc_scatter(D, True)
    scatE = _make_sc_scatter(ED, False)
    zrowN = jnp.zeros((8, D), f32)
    zrowE = jnp.zeros((8, ED), f32)
    Pn = jnp.zeros((N, D), f32)
    Pe = jnp.zeros((N, ED), f32)
    ea_t = edge_attr.T

    def gather_c(c, tok):
        lo = c * EC
        sidx3 = lax.dynamic_slice_in_dim(edge_index[0], lo, EC).reshape(NW, NG, G)
        tidx3 = lax.dynamic_slice_in_dim(edge_index[1], lo, EC).reshape(NW, NG, G)
        return _sc_gather(source, target, sidx3, tidx3, tok)

    S = [None] * CH
    T = [None] * CH
    S[0], T[0] = gather_c(0, zrowN)
    S[1], T[1] = gather_c(1, S[0])
    upds = []
    for c in range(CH):
        lo = c * EC
        tidx = lax.dynamic_slice_in_dim(edge_index[1], lo, EC)
        numer, ex_t, upd_t = _tc_edge(
            S[c], T[c], lax.dynamic_slice_in_dim(ea_t, lo, EC, axis=1), weights)
        tokn = S[c + 1] if c + 1 < CH else Pe
        Pn_new = scatN(numer, tidx, Pn, zrowN, tokn)
        Pe = scatE(ex_t.T, tidx, Pe, zrowE, Pn_new)
        Pn = Pn_new
        if c + 2 < CH:
            S[c + 2], T[c + 2] = gather_c(c + 2, Pe)
        upds.append(upd_t)
    y = _tc_node(Pn, Pe, target, params)
    upd = jnp.concatenate(upds, axis=1).T
    return (y, upd)


# speedup vs baseline: 1.5434x; 1.1865x over previous
import functools

import jax
import jax.numpy as jnp
from jax import lax
from jax.experimental import pallas as pl
from jax.experimental.pallas import tpu as pltpu
from jax.experimental.pallas import tpu_sc as plsc

N = 10000
E = 320000
D = 128
H = 4
DH = 32
ED = 16
EF = D + ED
HB = max(32, EF // 2)
FF = 4 * D
EUH = max(EF, D)

NC = 2
NS = 16
NW = NC * NS
CH = 5
EC = E // CH
G = 80
EPW = EC // NW
NG = EPW // G
EPT = EC // NS
NGS = EPT // G

HALF = N // NC
ACC_ROWS = HALF + 8

BE = 2560
BN = 2000


def _sc_gather(source, target, sidx3, tidx3, tok):
    mesh = plsc.VectorSubcoreMesh(core_axis_name="c", subcore_axis_name="s")

    @functools.partial(
        pl.kernel,
        out_type=(
            jax.ShapeDtypeStruct((EC, D), jnp.float32),
            jax.ShapeDtypeStruct((EC, D), jnp.float32),
        ),
        mesh=mesh,
        scratch_types=[
            pltpu.VMEM((NG, G), jnp.int32),
            pltpu.VMEM((NG, G), jnp.int32),
            pltpu.VMEM((2, G, D), jnp.float32),
            pltpu.VMEM((2, G, D), jnp.float32),
            pltpu.SemaphoreType.DMA,
            pltpu.SemaphoreType.DMA,
            pltpu.SemaphoreType.DMA,
            pltpu.SemaphoreType.DMA,
            pltpu.SemaphoreType.DMA,
            pltpu.SemaphoreType.DMA,
            pltpu.SemaphoreType.DMA,
            pltpu.SemaphoreType.DMA,
        ],
    )
    def k(src_hbm, tgt_hbm, sidx_hbm, tidx_hbm, tok_hbm, outS, outT,
          sidx_v, tidx_v, rowS, rowT, sS0, sS1, sT0, sT1, w0, w1, w2, w3):
        wid = lax.axis_index("s") * NC + lax.axis_index("c")
        base = wid * EPW
        pltpu.sync_copy(sidx_hbm.at[wid], sidx_v)
        pltpu.sync_copy(tidx_hbm.at[wid], tidx_v)

        def pair(j0, j1):
            gS0 = pltpu.async_copy(src_hbm.at[sidx_v.at[j0]], rowS.at[0], sS0)
            gT0 = pltpu.async_copy(tgt_hbm.at[tidx_v.at[j0]], rowT.at[0], sT0)
            gS1 = pltpu.async_copy(src_hbm.at[sidx_v.at[j1]], rowS.at[1], sS1)
            gT1 = pltpu.async_copy(tgt_hbm.at[tidx_v.at[j1]], rowT.at[1], sT1)
            gS0.wait()
            wS0 = pltpu.async_copy(rowS.at[0], outS.at[pl.ds(base + j0 * G, G)], w0)
            gT0.wait()
            wT0 = pltpu.async_copy(rowT.at[0], outT.at[pl.ds(base + j0 * G, G)], w1)
            gS1.wait()
            wS1 = pltpu.async_copy(rowS.at[1], outS.at[pl.ds(base + j1 * G, G)], w2)
            gT1.wait()
            wT1 = pltpu.async_copy(rowT.at[1], outT.at[pl.ds(base + j1 * G, G)], w3)
            wS0.wait()
            wT0.wait()
            wS1.wait()
            wT1.wait()

        def body(jj, carry):
            pair(jj * 2, jj * 2 + 1)
            return carry

        lax.fori_loop(0, NG // 2, body, 0)
        if NG % 2:
            j = NG - 1
            gS0 = pltpu.async_copy(src_hbm.at[sidx_v.at[j]], rowS.at[0], sS0)
            gT0 = pltpu.async_copy(tgt_hbm.at[tidx_v.at[j]], rowT.at[0], sT0)
            gS0.wait()
            pltpu.sync_copy(rowS.at[0], outS.at[pl.ds(base + j * G, G)])
            gT0.wait()
            pltpu.sync_copy(rowT.at[0], outT.at[pl.ds(base + j * G, G)])

    return k(source, target, sidx3, tidx3, tok)


NSL = 4


def _make_sc_scatter(W, tc_tiling):
    mesh = plsc.VectorSubcoreMesh(core_axis_name="c", subcore_axis_name="s")

    @functools.partial(
        pl.kernel,
        out_type=jax.ShapeDtypeStruct((N, W), jnp.float32),
        mesh=mesh,
        scratch_types=[
            pltpu.VMEM((NSL, G), jnp.int32),
            pltpu.VMEM((NSL, G), jnp.int32),
            pltpu.VMEM((NSL, G, W), jnp.float32),
            pltpu.VMEM_SHARED((ACC_ROWS, W), jnp.float32),
            [pltpu.SemaphoreType.DMA] * NSL,
            [pltpu.SemaphoreType.DMA] * NSL,
            [pltpu.SemaphoreType.DMA] * NSL,
        ],
        compiler_params=pltpu.CompilerParams(use_tc_tiling_on_sc=tc_tiling),
    )
    def k(pay_hbm, tidx_hbm, prev_hbm, zrow_hbm, tok_hbm, outP,
          idx_v, map_v, rowP, accP, sI, sP, sS):
        cid = lax.axis_index("c")
        sid = lax.axis_index("s")
        tbase = sid * EPT
        nbase = cid * HALF

        @pl.when(sid == 0)
        def _init():
            pltpu.sync_copy(prev_hbm.at[pl.ds(cid * HALF, HALF)],
                            accP.at[pl.ds(0, HALF)])
            pltpu.sync_copy(zrow_hbm, accP.at[pl.ds(HALF, 8)])

        plsc.subcore_barrier()

        def load(j, slot):
            lI = pltpu.async_copy(tidx_hbm.at[pl.ds(tbase + j * G, G)],
                                  idx_v.at[slot], sI[slot])
            lP = pltpu.async_copy(pay_hbm.at[pl.ds(tbase + j * G, G)],
                                  rowP.at[slot], sP[slot])
            return lI, lP

        def scat(j, slot, lI, lP):
            lI.wait()
            for kk in range(G // 16):
                v = idx_v[slot, pl.ds(kk * 16, 16)]
                loc = v - nbase
                inb = (loc >= 0) & (loc < HALF)
                map_v[slot, pl.ds(kk * 16, 16)] = jnp.where(inb, loc, HALF)
            lP.wait()
            return pltpu.async_copy(rowP.at[slot], accP.at[map_v.at[slot]],
                                    sS[slot], add=True)

        def group(j0, nsl):
            ls = [load(j0 + u, u) for u in range(nsl)]
            ws = [scat(j0 + u, u, *ls[u]) for u in range(nsl)]
            for w in ws:
                w.wait()

        def body(jj, carry):
            group(jj * NSL, NSL)
            return carry

        lax.fori_loop(0, NGS // NSL, body, 0)
        if NGS % NSL:
            group(NGS - NGS % NSL, NGS % NSL)
        plsc.subcore_barrier()

        @pl.when(sid == 0)
        def _dump():
            pltpu.sync_copy(accP.at[pl.ds(0, HALF)],
                            outP.at[pl.ds(cid * HALF, HALF)])

    return k


def _ln_rows(x, eps=1e-5):
    mu = jnp.mean(x, axis=-1, keepdims=True)
    v = jnp.mean(x * x, axis=-1, keepdims=True) - mu * mu
    return (x - mu) * lax.rsqrt(v + eps)


def _edge_body(S_ref, T_ref, EAT_ref,
               Wkv_ref, Wq_ref, Wen_ref, Wee_ref, be_ref,
               Wb1n_ref, Wb1e_ref, bb1_ref, Wb2_ref, bb2_ref,
               Wu1a_ref, Wu1b_ref, Wu1c_ref, bu1_ref, Wu2_ref, bu2_ref,
               gkv_ref, bkv_ref, gq_ref, bq_ref, gen_ref, ben_ref,
               it_ref, sel_ref, selt_ref,
               num_ref, ext_ref, updt_ref):
    f32 = jnp.float32
    s = S_ref[...]
    t = T_ref[...]
    ea = EAT_ref[...].T
    sn = _ln_rows(s) * gkv_ref[...] + bkv_ref[...]
    tn = _ln_rows(t) * gq_ref[...] + bq_ref[...]
    kv = jnp.dot(sn, Wkv_ref[...], preferred_element_type=f32)
    k_n = kv[:, :D]
    v_n = kv[:, D:]
    q = jnp.dot(tn, Wq_ref[...], preferred_element_type=f32)
    pw = s * t
    ssum = jnp.sum(pw, axis=-1, keepdims=True) + jnp.sum(ea, axis=-1, keepdims=True)
    ssq = jnp.sum(pw * pw, axis=-1, keepdims=True) + jnp.sum(ea * ea, axis=-1, keepdims=True)
    mu = ssum / EF
    var = ssq / EF - mu * mu
    inv = lax.rsqrt(var + 1e-5)
    efn = (pw - mu) * inv * gen_ref[:, :D] + ben_ref[:, :D]
    efe = (ea - mu) * inv * gen_ref[:, D:] + ben_ref[:, D:]
    kve = (jnp.dot(efn, Wen_ref[...], preferred_element_type=f32)
           + jnp.dot(efe, Wee_ref[...], preferred_element_type=f32)
           + be_ref[...])
    sk = k_n + kve[:, :D]
    sv = v_n + kve[:, D:]
    h = jax.nn.relu(jnp.dot(efn, Wb1n_ref[...], preferred_element_type=f32)
                    + jnp.dot(efe, Wb1e_ref[...], preferred_element_type=f32)
                    + bb1_ref[...])
    bias = jnp.dot(h, Wb2_ref[...], preferred_element_type=f32) + bb2_ref[...]
    prod = q * sk
    logits = (jnp.dot(prod, sel_ref[...], preferred_element_type=f32)
              * it_ref[...] + bias)
    ex = jnp.exp(logits)
    exb = jnp.dot(ex, selt_ref[...], preferred_element_type=f32)
    num_ref[...] = exb * sv
    ext_ref[...] = jnp.concatenate(
        [ex, jnp.zeros((ex.shape[0], ED - H), f32)], axis=-1).T
    g1 = jax.nn.relu(jnp.dot(efn, Wu1a_ref[...], preferred_element_type=f32)
                     + jnp.dot(efe, Wu1b_ref[...], preferred_element_type=f32)
                     + jnp.dot(sv, Wu1c_ref[...], preferred_element_type=f32)
                     + bu1_ref[...])
    updt_ref[...] = (jnp.dot(g1, Wu2_ref[...], preferred_element_type=f32)
                     + bu2_ref[...]).T


def _edge_weights(p):
    f32 = jnp.float32
    Wkv = jnp.concatenate([p["WkN"], p["WvN"]], axis=1)
    We = jnp.concatenate([p["WkE"], p["WvE"]], axis=1)
    Wen, Wee = We[:D], We[D:]
    be = jnp.concatenate([p["bkE"], p["bvE"]])[None, :]
    Wb1n, Wb1e = p["Wb1"][:D], p["Wb1"][D:]
    Wu1a, Wu1b, Wu1c = p["Weu1"][:D], p["Weu1"][D:EF], p["Weu1"][EF:]
    sel = (jnp.arange(D)[:, None] // DH == jnp.arange(H)[None, :]).astype(f32)
    selt = sel.T
    row = lambda v: v[None, :]
    return [
        Wkv, p["Wq"], Wen, Wee, be,
        Wb1n, Wb1e, row(p["bb1"]), p["Wb2"], row(p["bb2"]),
        Wu1a, Wu1b, Wu1c, row(p["beu1"]), p["Weu2"], row(p["beu2"]),
        row(p["g_kv"]), row(p["b_kv"]), row(p["g_q"]), row(p["b_q"]),
        row(p["g_en"]), row(p["b_en"]),
        row(p["inv_temp"]), sel, selt,
    ]


def _tc_edge(S, T, edge_attr_t, weights):
    f32 = jnp.float32
    full = lambda a: pl.BlockSpec(a.shape, lambda i: (0,) * a.ndim)
    grid = EC // BE
    return pl.pallas_call(
        _edge_body,
        grid=(grid,),
        in_specs=[
            pl.BlockSpec((BE, D), lambda i: (i, 0)),
            pl.BlockSpec((BE, D), lambda i: (i, 0)),
            pl.BlockSpec((ED, BE), lambda i: (0, i)),
        ] + [full(w) for w in weights],
        out_specs=[
            pl.BlockSpec((BE, D), lambda i: (i, 0)),
            pl.BlockSpec((ED, BE), lambda i: (0, i)),
            pl.BlockSpec((ED, BE), lambda i: (0, i)),
        ],
        out_shape=[
            jax.ShapeDtypeStruct((EC, D), f32),
            jax.ShapeDtypeStruct((ED, EC), f32),
            jax.ShapeDtypeStruct((ED, EC), f32),
        ],
        compiler_params=pltpu.CompilerParams(
            dimension_semantics=("arbitrary",),
        ),
    )(S, T, edge_attr_t, *weights)


def _erf(x):
    a1, a2, a3, a4, a5 = (0.254829592, -0.284496736, 1.421413741,
                          -1.453152027, 1.061405429)
    sgn = jnp.sign(x)
    ax = jnp.abs(x)
    t = 1.0 / (1.0 + 0.3275911 * ax)
    poly = ((((a5 * t + a4) * t + a3) * t + a2) * t + a1) * t
    return sgn * (1.0 - poly * jnp.exp(-ax * ax))


def _node_body(Pn_ref, Pe_ref, tgt_ref,
               Wout_ref, bout_ref, Wg_ref, bg_ref, Wu_ref, bu_ref,
               Wd_ref, bd_ref, g1_ref, b1_ref, g2_ref, b2_ref,
               sc_ref, selt_ref, y_ref):
    f32 = jnp.float32
    numer = Pn_ref[...]
    s16 = Pe_ref[...]
    sb = jnp.dot(s16, selt_ref[...], preferred_element_type=f32)
    att = numer / (sb + 1e-16)
    out = jnp.dot(att, Wout_ref[...], preferred_element_type=f32) + bout_ref[...]
    res_scale = sc_ref[0, 0]
    ffn_scale = sc_ref[0, 1]
    y = tgt_ref[...] + res_scale * out
    y = _ln_rows(y) * g1_ref[...] + b1_ref[...]
    gate = jnp.dot(y, Wg_ref[...], preferred_element_type=f32) + bg_ref[...]
    up = jnp.dot(y, Wu_ref[...], preferred_element_type=f32) + bu_ref[...]
    gelu = up * 0.5 * (1.0 + _erf(up * 0.7071067811865475))
    yff = jnp.dot(gate * gelu, Wd_ref[...], preferred_element_type=f32) + bd_ref[...]
    y = y + ffn_scale * yff
    y_ref[...] = _ln_rows(y) * g2_ref[...] + b2_ref[...]


def _tc_node(Pn, Pe, target, p):
    f32 = jnp.float32
    selt16 = (jnp.arange(ED)[:, None] == jnp.arange(D)[None, :] // DH).astype(f32)
    row = lambda v: v[None, :]
    scales = jnp.concatenate([p["res_scale"], p["ffn_scale"]])[None, :]
    weights = [
        p["Wout"], row(p["bout"]), p["Wg"], row(p["bg"]), p["Wu"], row(p["bu"]),
        p["Wd"], row(p["bd"]), row(p["g_1"]), row(p["b_1"]),
        row(p["g_2"]), row(p["b_2"]), scales, selt16,
    ]
    full = lambda a: pl.BlockSpec(a.shape, lambda i: (0,) * a.ndim)
    grid = N // BN
    return pl.pallas_call(
        _node_body,
        grid=(grid,),
        in_specs=(
            [pl.BlockSpec((BN, D), lambda i: (i, 0)),
             pl.BlockSpec((BN, ED), lambda i: (i, 0)),
             pl.BlockSpec((BN, D), lambda i: (i, 0))]
            + [full(w) for w in weights]
        ),
        out_specs=pl.BlockSpec((BN, D), lambda i: (i, 0)),
        out_shape=jax.ShapeDtypeStruct((N, D), f32),
        compiler_params=pltpu.CompilerParams(
            dimension_semantics=("arbitrary",),
        ),
    )(Pn, Pe, target, *weights)


def kernel(source, target, edge_index, edge_attr, params):
    f32 = jnp.float32
    weights = _edge_weights(params)
    scatN = _make_sc_scatter(D, True)
    scatE = _make_sc_scatter(ED, False)
    zrowN = jnp.zeros((8, D), f32)
    zrowE = jnp.zeros((8, ED), f32)
    Pn = jnp.zeros((N, D), f32)
    Pe = jnp.zeros((N, ED), f32)
    ea_t = edge_attr.T

    def gather_c(c, tok):
        lo = c * EC
        sidx3 = lax.dynamic_slice_in_dim(edge_index[0], lo, EC).reshape(NW, NG, G)
        tidx3 = lax.dynamic_slice_in_dim(edge_index[1], lo, EC).reshape(NW, NG, G)
        return _sc_gather(source, target, sidx3, tidx3, tok)

    S = [None] * CH
    T = [None] * CH
    for c in range(CH):
        S[c], T[c] = gather_c(c, zrowN)
    upds = []
    for c in range(CH):
        lo = c * EC
        tidx = lax.dynamic_slice_in_dim(edge_index[1], lo, EC)
        numer, ex_t, upd_t = _tc_edge(
            S[c], T[c], lax.dynamic_slice_in_dim(ea_t, lo, EC, axis=1), weights)
        Pn = scatN(numer, tidx, Pn, zrowN, zrowN)
        Pe = scatE(ex_t.T, tidx, Pe, zrowE, zrowE)
        upds.append(upd_t)
    y = _tc_node(Pn, Pe, target, params)
    upd = jnp.concatenate(upds, axis=1).T
    return (y, upd)


# speedup vs baseline: 1.6946x; 1.0980x over previous
import functools

import jax
import jax.numpy as jnp
from jax import lax
from jax.experimental import pallas as pl
from jax.experimental.pallas import tpu as pltpu
from jax.experimental.pallas import tpu_sc as plsc

N = 10000
E = 320000
D = 128
H = 4
DH = 32
ED = 16
EF = D + ED
HB = max(32, EF // 2)
FF = 4 * D
EUH = max(EF, D)

NC = 2
NS = 16
NW = NC * NS
CH = 5
EC = E // CH
G = 80
EPW = EC // NW
NG = EPW // G
EPT = EC // NS
NGS = EPT // G

HALF = N // NC
ACC_ROWS = HALF + 8

BE = 2560
BN = 2000


def _sc_gather(source, target, sidx3, tidx3, tok):
    mesh = plsc.VectorSubcoreMesh(core_axis_name="c", subcore_axis_name="s")

    @functools.partial(
        pl.kernel,
        out_type=(
            jax.ShapeDtypeStruct((EC, D), jnp.float32),
            jax.ShapeDtypeStruct((EC, D), jnp.float32),
        ),
        mesh=mesh,
        scratch_types=[
            pltpu.VMEM((NG, G), jnp.int32),
            pltpu.VMEM((NG, G), jnp.int32),
            pltpu.VMEM((2, G, D), jnp.float32),
            pltpu.VMEM((2, G, D), jnp.float32),
            pltpu.SemaphoreType.DMA,
            pltpu.SemaphoreType.DMA,
            pltpu.SemaphoreType.DMA,
            pltpu.SemaphoreType.DMA,
            pltpu.SemaphoreType.DMA,
            pltpu.SemaphoreType.DMA,
            pltpu.SemaphoreType.DMA,
            pltpu.SemaphoreType.DMA,
        ],
    )
    def k(src_hbm, tgt_hbm, sidx_hbm, tidx_hbm, tok_hbm, outS, outT,
          sidx_v, tidx_v, rowS, rowT, sS0, sS1, sT0, sT1, w0, w1, w2, w3):
        wid = lax.axis_index("s") * NC + lax.axis_index("c")
        base = wid * EPW
        pltpu.sync_copy(sidx_hbm.at[wid], sidx_v)
        pltpu.sync_copy(tidx_hbm.at[wid], tidx_v)

        def pair(j0, j1):
            gS0 = pltpu.async_copy(src_hbm.at[sidx_v.at[j0]], rowS.at[0], sS0)
            gT0 = pltpu.async_copy(tgt_hbm.at[tidx_v.at[j0]], rowT.at[0], sT0)
            gS1 = pltpu.async_copy(src_hbm.at[sidx_v.at[j1]], rowS.at[1], sS1)
            gT1 = pltpu.async_copy(tgt_hbm.at[tidx_v.at[j1]], rowT.at[1], sT1)
            gS0.wait()
            wS0 = pltpu.async_copy(rowS.at[0], outS.at[pl.ds(base + j0 * G, G)], w0)
            gT0.wait()
            wT0 = pltpu.async_copy(rowT.at[0], outT.at[pl.ds(base + j0 * G, G)], w1)
            gS1.wait()
            wS1 = pltpu.async_copy(rowS.at[1], outS.at[pl.ds(base + j1 * G, G)], w2)
            gT1.wait()
            wT1 = pltpu.async_copy(rowT.at[1], outT.at[pl.ds(base + j1 * G, G)], w3)
            wS0.wait()
            wT0.wait()
            wS1.wait()
            wT1.wait()

        def body(jj, carry):
            pair(jj * 2, jj * 2 + 1)
            return carry

        lax.fori_loop(0, NG // 2, body, 0)
        if NG % 2:
            j = NG - 1
            gS0 = pltpu.async_copy(src_hbm.at[sidx_v.at[j]], rowS.at[0], sS0)
            gT0 = pltpu.async_copy(tgt_hbm.at[tidx_v.at[j]], rowT.at[0], sT0)
            gS0.wait()
            pltpu.sync_copy(rowS.at[0], outS.at[pl.ds(base + j * G, G)])
            gT0.wait()
            pltpu.sync_copy(rowT.at[0], outT.at[pl.ds(base + j * G, G)])

    return k(source, target, sidx3, tidx3, tok)


NSL = 4


def _make_sc_scatter(W, tc_tiling):
    mesh = plsc.VectorSubcoreMesh(core_axis_name="c", subcore_axis_name="s")

    @functools.partial(
        pl.kernel,
        out_type=jax.ShapeDtypeStruct((N, W), jnp.float32),
        mesh=mesh,
        scratch_types=[
            pltpu.VMEM((NSL, G), jnp.int32),
            pltpu.VMEM((NSL, G), jnp.int32),
            pltpu.VMEM((NSL, G, W), jnp.float32),
            pltpu.VMEM_SHARED((ACC_ROWS, W), jnp.float32),
            [pltpu.SemaphoreType.DMA] * NSL,
            [pltpu.SemaphoreType.DMA] * NSL,
            [pltpu.SemaphoreType.DMA] * NSL,
        ],
        compiler_params=pltpu.CompilerParams(use_tc_tiling_on_sc=tc_tiling),
    )
    def k(pay_hbm, tidx_hbm, prev_hbm, zrow_hbm, tok_hbm, outP,
          idx_v, map_v, rowP, accP, sI, sP, sS):
        cid = lax.axis_index("c")
        sid = lax.axis_index("s")
        tbase = sid * EPT
        nbase = cid * HALF

        @pl.when(sid == 0)
        def _init():
            pltpu.sync_copy(prev_hbm.at[pl.ds(cid * HALF, HALF)],
                            accP.at[pl.ds(0, HALF)])
            pltpu.sync_copy(zrow_hbm, accP.at[pl.ds(HALF, 8)])

        plsc.subcore_barrier()

        def load(j, slot):
            lI = pltpu.async_copy(tidx_hbm.at[pl.ds(tbase + j * G, G)],
                                  idx_v.at[slot], sI[slot])
            lP = pltpu.async_copy(pay_hbm.at[pl.ds(tbase + j * G, G)],
                                  rowP.at[slot], sP[slot])
            return lI, lP

        def scat(j, slot, lI, lP):
            lI.wait()
            for kk in range(G // 16):
                v = idx_v[slot, pl.ds(kk * 16, 16)]
                loc = v - nbase
                inb = (loc >= 0) & (loc < HALF)
                map_v[slot, pl.ds(kk * 16, 16)] = jnp.where(inb, loc, HALF)
            lP.wait()
            return pltpu.async_copy(rowP.at[slot], accP.at[map_v.at[slot]],
                                    sS[slot], add=True)

        def group(j0, nsl):
            ls = [load(j0 + u, u) for u in range(nsl)]
            ws = [scat(j0 + u, u, *ls[u]) for u in range(nsl)]
            for w in ws:
                w.wait()

        def body(jj, carry):
            group(jj * NSL, NSL)
            return carry

        lax.fori_loop(0, NGS // NSL, body, 0)
        if NGS % NSL:
            group(NGS - NGS % NSL, NGS % NSL)
        plsc.subcore_barrier()

        @pl.when(sid == 0)
        def _dump():
            pltpu.sync_copy(accP.at[pl.ds(0, HALF)],
                            outP.at[pl.ds(cid * HALF, HALF)])

    return k


def _ln_rows(x, eps=1e-5):
    mu = jnp.mean(x, axis=-1, keepdims=True)
    v = jnp.mean(x * x, axis=-1, keepdims=True) - mu * mu
    return (x - mu) * lax.rsqrt(v + eps)


def _edge_body(S_ref, T_ref, EAT_ref,
               Wkv_ref, Wq_ref, Wen_ref, Wee_ref, be_ref,
               Wb1n_ref, Wb1e_ref, bb1_ref, Wb2_ref, bb2_ref,
               Wu1a_ref, Wu1b_ref, Wu1c_ref, bu1_ref, Wu2_ref, bu2_ref,
               gkv_ref, bkv_ref, gq_ref, bq_ref, gen_ref, ben_ref,
               it_ref, sel_ref, selt_ref, rep_ref, grp_ref,
               num_ref, ext_ref, updt_ref):
    f32 = jnp.float32
    s = S_ref[...]
    t = T_ref[...]
    ea = EAT_ref[...].T
    sn = _ln_rows(s) * gkv_ref[...] + bkv_ref[...]
    tn = _ln_rows(t) * gq_ref[...] + bq_ref[...]
    kv = jnp.dot(sn, Wkv_ref[...], preferred_element_type=f32)
    k_n = kv[:, :D]
    v_n = kv[:, D:]
    q = jnp.dot(tn, Wq_ref[...], preferred_element_type=f32)
    pw = s * t
    ssum = jnp.sum(pw, axis=-1, keepdims=True) + jnp.sum(ea, axis=-1, keepdims=True)
    ssq = jnp.sum(pw * pw, axis=-1, keepdims=True) + jnp.sum(ea * ea, axis=-1, keepdims=True)
    mu = ssum / EF
    var = ssq / EF - mu * mu
    inv = lax.rsqrt(var + 1e-5)
    efn = (pw - mu) * inv * gen_ref[:, :D] + ben_ref[:, :D]
    efe = (ea - mu) * inv * gen_ref[:, D:] + ben_ref[:, D:]
    kve = (jnp.dot(efn, Wen_ref[...], preferred_element_type=f32)
           + jnp.dot(efe, Wee_ref[...], preferred_element_type=f32)
           + be_ref[...])
    sk = k_n + kve[:, :D]
    sv = v_n + kve[:, D:]
    h = jax.nn.relu(jnp.dot(efn, Wb1n_ref[...], preferred_element_type=f32)
                    + jnp.dot(efe, Wb1e_ref[...], preferred_element_type=f32)
                    + bb1_ref[...])
    bias = jnp.dot(h, Wb2_ref[...], preferred_element_type=f32) + bb2_ref[...]
    prod = q * sk
    logits = (jnp.dot(prod, sel_ref[...], preferred_element_type=f32)
              * it_ref[...] + bias)
    ex = jnp.exp(logits)
    exb = jnp.dot(ex, selt_ref[...], preferred_element_type=f32)
    num_ref[...] = exb * sv
    exR = jnp.dot(ex, rep_ref[...], preferred_element_type=f32)
    slot_row = lax.broadcasted_iota(jnp.int32, (ex.shape[0], 1), 0) % 8
    lane_slot = lax.broadcasted_iota(jnp.int32, (1, 128), 1) // ED
    Y = jnp.where(lane_slot == slot_row, exR, 0.0)
    ext_ref[...] = jnp.dot(grp_ref[...], Y, preferred_element_type=f32)
    g1 = jax.nn.relu(jnp.dot(efn, Wu1a_ref[...], preferred_element_type=f32)
                     + jnp.dot(efe, Wu1b_ref[...], preferred_element_type=f32)
                     + jnp.dot(sv, Wu1c_ref[...], preferred_element_type=f32)
                     + bu1_ref[...])
    updt_ref[...] = (jnp.dot(g1, Wu2_ref[...], preferred_element_type=f32)
                     + bu2_ref[...]).T


def _edge_weights(p):
    f32 = jnp.float32
    Wkv = jnp.concatenate([p["WkN"], p["WvN"]], axis=1)
    We = jnp.concatenate([p["WkE"], p["WvE"]], axis=1)
    Wen, Wee = We[:D], We[D:]
    be = jnp.concatenate([p["bkE"], p["bvE"]])[None, :]
    Wb1n, Wb1e = p["Wb1"][:D], p["Wb1"][D:]
    Wu1a, Wu1b, Wu1c = p["Weu1"][:D], p["Weu1"][D:EF], p["Weu1"][EF:]
    sel = (jnp.arange(D)[:, None] // DH == jnp.arange(H)[None, :]).astype(f32)
    selt = sel.T
    row = lambda v: v[None, :]
    return [
        Wkv, p["Wq"], Wen, Wee, be,
        Wb1n, Wb1e, row(p["bb1"]), p["Wb2"], row(p["bb2"]),
        Wu1a, Wu1b, Wu1c, row(p["beu1"]), p["Weu2"], row(p["beu2"]),
        row(p["g_kv"]), row(p["b_kv"]), row(p["g_q"]), row(p["b_q"]),
        row(p["g_en"]), row(p["b_en"]),
        row(p["inv_temp"]), sel, selt,
        (jnp.arange(H)[:, None] == jnp.arange(D)[None, :] % ED).astype(f32),
        (jnp.arange(BE // 8)[:, None] == jnp.arange(BE)[None, :] // 8).astype(f32),
    ]


def _tc_edge(S, T, edge_attr_t, weights):
    f32 = jnp.float32
    full = lambda a: pl.BlockSpec(a.shape, lambda i: (0,) * a.ndim)
    grid = EC // BE
    return pl.pallas_call(
        _edge_body,
        grid=(grid,),
        in_specs=[
            pl.BlockSpec((BE, D), lambda i: (i, 0)),
            pl.BlockSpec((BE, D), lambda i: (i, 0)),
            pl.BlockSpec((ED, BE), lambda i: (0, i)),
        ] + [full(w) for w in weights],
        out_specs=[
            pl.BlockSpec((BE, D), lambda i: (i, 0)),
            pl.BlockSpec((BE // 8, 128), lambda i: (i, 0)),
            pl.BlockSpec((ED, BE), lambda i: (0, i)),
        ],
        out_shape=[
            jax.ShapeDtypeStruct((EC, D), f32),
            jax.ShapeDtypeStruct((EC // 8, 128), f32),
            jax.ShapeDtypeStruct((ED, EC), f32),
        ],
        compiler_params=pltpu.CompilerParams(
            dimension_semantics=("arbitrary",),
        ),
    )(S, T, edge_attr_t, *weights)


def _erf(x):
    a1, a2, a3, a4, a5 = (0.254829592, -0.284496736, 1.421413741,
                          -1.453152027, 1.061405429)
    sgn = jnp.sign(x)
    ax = jnp.abs(x)
    t = 1.0 / (1.0 + 0.3275911 * ax)
    poly = ((((a5 * t + a4) * t + a3) * t + a2) * t + a1) * t
    return sgn * (1.0 - poly * jnp.exp(-ax * ax))


def _node_body(Pn_ref, Pe_ref, tgt_ref,
               Wout_ref, bout_ref, Wg_ref, bg_ref, Wu_ref, bu_ref,
               Wd_ref, bd_ref, g1_ref, b1_ref, g2_ref, b2_ref,
               sc_ref, selt_ref, y_ref):
    f32 = jnp.float32
    numer = Pn_ref[...]
    s16 = Pe_ref[...]
    sb = jnp.dot(s16, selt_ref[...], preferred_element_type=f32)
    att = numer / (sb + 1e-16)
    out = jnp.dot(att, Wout_ref[...], preferred_element_type=f32) + bout_ref[...]
    res_scale = sc_ref[0, 0]
    ffn_scale = sc_ref[0, 1]
    y = tgt_ref[...] + res_scale * out
    y = _ln_rows(y) * g1_ref[...] + b1_ref[...]
    gate = jnp.dot(y, Wg_ref[...], preferred_element_type=f32) + bg_ref[...]
    up = jnp.dot(y, Wu_ref[...], preferred_element_type=f32) + bu_ref[...]
    gelu = up * 0.5 * (1.0 + _erf(up * 0.7071067811865475))
    yff = jnp.dot(gate * gelu, Wd_ref[...], preferred_element_type=f32) + bd_ref[...]
    y = y + ffn_scale * yff
    y_ref[...] = _ln_rows(y) * g2_ref[...] + b2_ref[...]


def _tc_node(Pn, Pe, target, p):
    f32 = jnp.float32
    selt16 = (jnp.arange(ED)[:, None] == jnp.arange(D)[None, :] // DH).astype(f32)
    row = lambda v: v[None, :]
    scales = jnp.concatenate([p["res_scale"], p["ffn_scale"]])[None, :]
    weights = [
        p["Wout"], row(p["bout"]), p["Wg"], row(p["bg"]), p["Wu"], row(p["bu"]),
        p["Wd"], row(p["bd"]), row(p["g_1"]), row(p["b_1"]),
        row(p["g_2"]), row(p["b_2"]), scales, selt16,
    ]
    full = lambda a: pl.BlockSpec(a.shape, lambda i: (0,) * a.ndim)
    grid = N // BN
    return pl.pallas_call(
        _node_body,
        grid=(grid,),
        in_specs=(
            [pl.BlockSpec((BN, D), lambda i: (i, 0)),
             pl.BlockSpec((BN, ED), lambda i: (i, 0)),
             pl.BlockSpec((BN, D), lambda i: (i, 0))]
            + [full(w) for w in weights]
        ),
        out_specs=pl.BlockSpec((BN, D), lambda i: (i, 0)),
        out_shape=jax.ShapeDtypeStruct((N, D), f32),
        compiler_params=pltpu.CompilerParams(
            dimension_semantics=("arbitrary",),
        ),
    )(Pn, Pe, target, *weights)


def kernel(source, target, edge_index, edge_attr, params):
    f32 = jnp.float32
    weights = _edge_weights(params)
    scatN = _make_sc_scatter(D, True)
    scatE = _make_sc_scatter(ED, False)
    zrowN = jnp.zeros((8, D), f32)
    zrowE = jnp.zeros((8, ED), f32)
    Pn = jnp.zeros((N, D), f32)
    Pe = jnp.zeros((N, ED), f32)
    ea_t = edge_attr.T

    def gather_c(c, tok):
        lo = c * EC
        sidx3 = lax.dynamic_slice_in_dim(edge_index[0], lo, EC).reshape(NW, NG, G)
        tidx3 = lax.dynamic_slice_in_dim(edge_index[1], lo, EC).reshape(NW, NG, G)
        return _sc_gather(source, target, sidx3, tidx3, tok)

    S = [None] * CH
    T = [None] * CH
    for c in range(CH):
        S[c], T[c] = gather_c(c, zrowN)
    upds = []
    for c in range(CH):
        lo = c * EC
        tidx = lax.dynamic_slice_in_dim(edge_index[1], lo, EC)
        numer, ex_pk, upd_t = _tc_edge(
            S[c], T[c], lax.dynamic_slice_in_dim(ea_t, lo, EC, axis=1), weights)
        Pn = scatN(numer, tidx, Pn, zrowN, zrowN)
        Pe = scatE(ex_pk.reshape(EC, ED), tidx, Pe, zrowE, zrowE)
        upds.append(upd_t)
    y = _tc_node(Pn, Pe, target, params)
    upd = jnp.concatenate(upds, axis=1).T
    return (y, upd)


# speedup vs baseline: 1.7442x; 1.0292x over previous
import functools

import jax
import jax.numpy as jnp
from jax import lax
from jax.experimental import pallas as pl
from jax.experimental.pallas import tpu as pltpu
from jax.experimental.pallas import tpu_sc as plsc

N = 10000
E = 320000
D = 128
H = 4
DH = 32
ED = 16
EF = D + ED
HB = max(32, EF // 2)
FF = 4 * D
EUH = max(EF, D)

NC = 2
NS = 16
NW = NC * NS
CH = 5
EC = E // CH
G = 80
EPW = EC // NW
NG = EPW // G
EPT = EC // NS
NGS = EPT // G

HALF = N // NC
ACC_ROWS = HALF + 8

BE = 3200
BN = 2000


def _sc_gather(source, target, sidx3, tidx3, tok):
    mesh = plsc.VectorSubcoreMesh(core_axis_name="c", subcore_axis_name="s")

    @functools.partial(
        pl.kernel,
        out_type=(
            jax.ShapeDtypeStruct((EC, D), jnp.float32),
            jax.ShapeDtypeStruct((EC, D), jnp.float32),
        ),
        mesh=mesh,
        scratch_types=[
            pltpu.VMEM((NG, G), jnp.int32),
            pltpu.VMEM((NG, G), jnp.int32),
            pltpu.VMEM((2, G, D), jnp.float32),
            pltpu.VMEM((2, G, D), jnp.float32),
            pltpu.SemaphoreType.DMA,
            pltpu.SemaphoreType.DMA,
            pltpu.SemaphoreType.DMA,
            pltpu.SemaphoreType.DMA,
            pltpu.SemaphoreType.DMA,
            pltpu.SemaphoreType.DMA,
            pltpu.SemaphoreType.DMA,
            pltpu.SemaphoreType.DMA,
        ],
    )
    def k(src_hbm, tgt_hbm, sidx_hbm, tidx_hbm, tok_hbm, outS, outT,
          sidx_v, tidx_v, rowS, rowT, sS0, sS1, sT0, sT1, w0, w1, w2, w3):
        wid = lax.axis_index("s") * NC + lax.axis_index("c")
        base = wid * EPW
        pltpu.sync_copy(sidx_hbm.at[wid], sidx_v)
        pltpu.sync_copy(tidx_hbm.at[wid], tidx_v)

        def pair(j0, j1):
            gS0 = pltpu.async_copy(src_hbm.at[sidx_v.at[j0]], rowS.at[0], sS0)
            gT0 = pltpu.async_copy(tgt_hbm.at[tidx_v.at[j0]], rowT.at[0], sT0)
            gS1 = pltpu.async_copy(src_hbm.at[sidx_v.at[j1]], rowS.at[1], sS1)
            gT1 = pltpu.async_copy(tgt_hbm.at[tidx_v.at[j1]], rowT.at[1], sT1)
            gS0.wait()
            wS0 = pltpu.async_copy(rowS.at[0], outS.at[pl.ds(base + j0 * G, G)], w0)
            gT0.wait()
            wT0 = pltpu.async_copy(rowT.at[0], outT.at[pl.ds(base + j0 * G, G)], w1)
            gS1.wait()
            wS1 = pltpu.async_copy(rowS.at[1], outS.at[pl.ds(base + j1 * G, G)], w2)
            gT1.wait()
            wT1 = pltpu.async_copy(rowT.at[1], outT.at[pl.ds(base + j1 * G, G)], w3)
            wS0.wait()
            wT0.wait()
            wS1.wait()
            wT1.wait()

        def body(jj, carry):
            pair(jj * 2, jj * 2 + 1)
            return carry

        lax.fori_loop(0, NG // 2, body, 0)
        if NG % 2:
            j = NG - 1
            gS0 = pltpu.async_copy(src_hbm.at[sidx_v.at[j]], rowS.at[0], sS0)
            gT0 = pltpu.async_copy(tgt_hbm.at[tidx_v.at[j]], rowT.at[0], sT0)
            gS0.wait()
            pltpu.sync_copy(rowS.at[0], outS.at[pl.ds(base + j * G, G)])
            gT0.wait()
            pltpu.sync_copy(rowT.at[0], outT.at[pl.ds(base + j * G, G)])

    return k(source, target, sidx3, tidx3, tok)


NSL = 5


def _make_sc_scatter(W, tc_tiling):
    mesh = plsc.VectorSubcoreMesh(core_axis_name="c", subcore_axis_name="s")

    @functools.partial(
        pl.kernel,
        out_type=jax.ShapeDtypeStruct((N, W), jnp.float32),
        mesh=mesh,
        scratch_types=[
            pltpu.VMEM((NSL, G), jnp.int32),
            pltpu.VMEM((NSL, G), jnp.int32),
            pltpu.VMEM((NSL, G, W), jnp.float32),
            pltpu.VMEM_SHARED((ACC_ROWS, W), jnp.float32),
            [pltpu.SemaphoreType.DMA] * NSL,
            [pltpu.SemaphoreType.DMA] * NSL,
            [pltpu.SemaphoreType.DMA] * NSL,
        ],
        compiler_params=pltpu.CompilerParams(use_tc_tiling_on_sc=tc_tiling),
    )
    def k(pay_hbm, tidx_hbm, prev_hbm, zrow_hbm, tok_hbm, outP,
          idx_v, map_v, rowP, accP, sI, sP, sS):
        cid = lax.axis_index("c")
        sid = lax.axis_index("s")
        tbase = sid * EPT
        nbase = cid * HALF

        @pl.when(sid == 0)
        def _init():
            pltpu.sync_copy(prev_hbm.at[pl.ds(cid * HALF, HALF)],
                            accP.at[pl.ds(0, HALF)])
            pltpu.sync_copy(zrow_hbm, accP.at[pl.ds(HALF, 8)])

        plsc.subcore_barrier()

        def load(j, slot):
            lI = pltpu.async_copy(tidx_hbm.at[pl.ds(tbase + j * G, G)],
                                  idx_v.at[slot], sI[slot])
            lP = pltpu.async_copy(pay_hbm.at[pl.ds(tbase + j * G, G)],
                                  rowP.at[slot], sP[slot])
            return lI, lP

        def scat(j, slot, lI, lP):
            lI.wait()
            for kk in range(G // 16):
                v = idx_v[slot, pl.ds(kk * 16, 16)]
                loc = v - nbase
                inb = (loc >= 0) & (loc < HALF)
                map_v[slot, pl.ds(kk * 16, 16)] = jnp.where(inb, loc, HALF)
            lP.wait()
            return pltpu.async_copy(rowP.at[slot], accP.at[map_v.at[slot]],
                                    sS[slot], add=True)

        def group(j0, nsl):
            ls = [load(j0 + u, u) for u in range(nsl)]
            ws = [scat(j0 + u, u, *ls[u]) for u in range(nsl)]
            for w in ws:
                w.wait()

        def body(jj, carry):
            group(jj * NSL, NSL)
            return carry

        lax.fori_loop(0, NGS // NSL, body, 0)
        if NGS % NSL:
            group(NGS - NGS % NSL, NGS % NSL)
        plsc.subcore_barrier()

        @pl.when(sid == 0)
        def _dump():
            pltpu.sync_copy(accP.at[pl.ds(0, HALF)],
                            outP.at[pl.ds(cid * HALF, HALF)])

    return k


def _sc_scatter_ex(payload, tidx, prev, tok):
    mesh = plsc.VectorSubcoreMesh(core_axis_name="c", subcore_axis_name="s")

    @functools.partial(
        pl.kernel,
        out_type=jax.ShapeDtypeStruct((NC, N, ED), jnp.float32),
        mesh=mesh,
        scratch_types=[
            pltpu.VMEM((NSL, G), jnp.int32),
            pltpu.VMEM((NSL, G, ED), jnp.float32),
            pltpu.VMEM_SHARED((N, ED), jnp.float32),
            [pltpu.SemaphoreType.DMA] * NSL,
            [pltpu.SemaphoreType.DMA] * NSL,
            [pltpu.SemaphoreType.DMA] * NSL,
        ],
        compiler_params=pltpu.CompilerParams(use_tc_tiling_on_sc=False),
    )
    def k(pay_hbm, tidx_hbm, prev_hbm, tok_hbm, outP,
          idx_v, rowP, accP, sI, sP, sS):
        cid = lax.axis_index("c")
        sid = lax.axis_index("s")
        wid = sid * NC + cid
        tbase = wid * EPW

        @pl.when(sid == 0)
        def _init():
            pltpu.sync_copy(prev_hbm.at[cid], accP)

        plsc.subcore_barrier()

        def load(j, slot):
            lI = pltpu.async_copy(tidx_hbm.at[pl.ds(tbase + j * G, G)],
                                  idx_v.at[slot], sI[slot])
            lP = pltpu.async_copy(pay_hbm.at[pl.ds(tbase + j * G, G)],
                                  rowP.at[slot], sP[slot])
            return lI, lP

        def scat(slot, lI, lP):
            lI.wait()
            lP.wait()
            return pltpu.async_copy(rowP.at[slot], accP.at[idx_v.at[slot]],
                                    sS[slot], add=True)

        def body(jj, carry):
            ls = [load(jj * NSL + u, u) for u in range(NSL)]
            ws = [scat(u, *ls[u]) for u in range(NSL)]
            for w in ws:
                w.wait()
            return carry

        lax.fori_loop(0, NG // NSL, body, 0)
        plsc.subcore_barrier()

        @pl.when(sid == 0)
        def _dump():
            pltpu.sync_copy(accP, outP.at[cid])

    return k(payload, tidx, prev, tok)


def _ln_rows(x, eps=1e-5):
    mu = jnp.mean(x, axis=-1, keepdims=True)
    v = jnp.mean(x * x, axis=-1, keepdims=True) - mu * mu
    return (x - mu) * lax.rsqrt(v + eps)


def _edge_body(S_ref, T_ref, EAT_ref,
               Wkv_ref, Wq_ref, Wen_ref, Wee_ref, be_ref,
               Wb1n_ref, Wb1e_ref, bb1_ref, Wb2_ref, bb2_ref,
               Wu1a_ref, Wu1b_ref, Wu1c_ref, bu1_ref, Wu2_ref, bu2_ref,
               gkv_ref, bkv_ref, gq_ref, bq_ref, gen_ref, ben_ref,
               it_ref, sel_ref, selt_ref, rep_ref, grp_ref,
               num_ref, ext_ref, updt_ref):
    f32 = jnp.float32
    s = S_ref[...]
    t = T_ref[...]
    ea = EAT_ref[...].T
    sn = _ln_rows(s) * gkv_ref[...] + bkv_ref[...]
    tn = _ln_rows(t) * gq_ref[...] + bq_ref[...]
    kv = jnp.dot(sn, Wkv_ref[...], preferred_element_type=f32)
    k_n = kv[:, :D]
    v_n = kv[:, D:]
    q = jnp.dot(tn, Wq_ref[...], preferred_element_type=f32)
    pw = s * t
    ssum = jnp.sum(pw, axis=-1, keepdims=True) + jnp.sum(ea, axis=-1, keepdims=True)
    ssq = jnp.sum(pw * pw, axis=-1, keepdims=True) + jnp.sum(ea * ea, axis=-1, keepdims=True)
    mu = ssum / EF
    var = ssq / EF - mu * mu
    inv = lax.rsqrt(var + 1e-5)
    efn = (pw - mu) * inv * gen_ref[:, :D] + ben_ref[:, :D]
    efe = (ea - mu) * inv * gen_ref[:, D:] + ben_ref[:, D:]
    kve = (jnp.dot(efn, Wen_ref[...], preferred_element_type=f32)
           + jnp.dot(efe, Wee_ref[...], preferred_element_type=f32)
           + be_ref[...])
    sk = k_n + kve[:, :D]
    sv = v_n + kve[:, D:]
    h = jax.nn.relu(jnp.dot(efn, Wb1n_ref[...], preferred_element_type=f32)
                    + jnp.dot(efe, Wb1e_ref[...], preferred_element_type=f32)
                    + bb1_ref[...])
    bias = jnp.dot(h, Wb2_ref[...], preferred_element_type=f32) + bb2_ref[...]
    prod = q * sk
    logits = (jnp.dot(prod, sel_ref[...], preferred_element_type=f32)
              * it_ref[...] + bias)
    ex = jnp.exp(logits)
    exb = jnp.dot(ex, selt_ref[...], preferred_element_type=f32)
    num_ref[...] = exb * sv
    exR = jnp.dot(ex, rep_ref[...], preferred_element_type=f32)
    slot_row = lax.broadcasted_iota(jnp.int32, (ex.shape[0], 1), 0) % 8
    lane_slot = lax.broadcasted_iota(jnp.int32, (1, 128), 1) // ED
    Y = jnp.where(lane_slot == slot_row, exR, 0.0)
    ext_ref[...] = jnp.dot(grp_ref[...], Y, preferred_element_type=f32)
    g1 = jax.nn.relu(jnp.dot(efn, Wu1a_ref[...], preferred_element_type=f32)
                     + jnp.dot(efe, Wu1b_ref[...], preferred_element_type=f32)
                     + jnp.dot(sv, Wu1c_ref[...], preferred_element_type=f32)
                     + bu1_ref[...])
    updt_ref[...] = (jnp.dot(g1, Wu2_ref[...], preferred_element_type=f32)
                     + bu2_ref[...]).T


def _edge_weights(p):
    f32 = jnp.float32
    Wkv = jnp.concatenate([p["WkN"], p["WvN"]], axis=1)
    We = jnp.concatenate([p["WkE"], p["WvE"]], axis=1)
    Wen, Wee = We[:D], We[D:]
    be = jnp.concatenate([p["bkE"], p["bvE"]])[None, :]
    Wb1n, Wb1e = p["Wb1"][:D], p["Wb1"][D:]
    Wu1a, Wu1b, Wu1c = p["Weu1"][:D], p["Weu1"][D:EF], p["Weu1"][EF:]
    sel = (jnp.arange(D)[:, None] // DH == jnp.arange(H)[None, :]).astype(f32)
    selt = sel.T
    row = lambda v: v[None, :]
    return [
        Wkv, p["Wq"], Wen, Wee, be,
        Wb1n, Wb1e, row(p["bb1"]), p["Wb2"], row(p["bb2"]),
        Wu1a, Wu1b, Wu1c, row(p["beu1"]), p["Weu2"], row(p["beu2"]),
        row(p["g_kv"]), row(p["b_kv"]), row(p["g_q"]), row(p["b_q"]),
        row(p["g_en"]), row(p["b_en"]),
        row(p["inv_temp"]), sel, selt,
        (jnp.arange(H)[:, None] == jnp.arange(D)[None, :] % ED).astype(f32),
        (jnp.arange(BE // 8)[:, None] == jnp.arange(BE)[None, :] // 8).astype(f32),
    ]


def _tc_edge(S, T, edge_attr_t, weights):
    f32 = jnp.float32
    full = lambda a: pl.BlockSpec(a.shape, lambda i: (0,) * a.ndim)
    grid = EC // BE
    return pl.pallas_call(
        _edge_body,
        grid=(grid,),
        in_specs=[
            pl.BlockSpec((BE, D), lambda i: (i, 0)),
            pl.BlockSpec((BE, D), lambda i: (i, 0)),
            pl.BlockSpec((ED, BE), lambda i: (0, i)),
        ] + [full(w) for w in weights],
        out_specs=[
            pl.BlockSpec((BE, D), lambda i: (i, 0)),
            pl.BlockSpec((BE // 8, 128), lambda i: (i, 0)),
            pl.BlockSpec((ED, BE), lambda i: (0, i)),
        ],
        out_shape=[
            jax.ShapeDtypeStruct((EC, D), f32),
            jax.ShapeDtypeStruct((EC // 8, 128), f32),
            jax.ShapeDtypeStruct((ED, EC), f32),
        ],
        compiler_params=pltpu.CompilerParams(
            dimension_semantics=("arbitrary",),
        ),
    )(S, T, edge_attr_t, *weights)


def _erf(x):
    a1, a2, a3, a4, a5 = (0.254829592, -0.284496736, 1.421413741,
                          -1.453152027, 1.061405429)
    sgn = jnp.sign(x)
    ax = jnp.abs(x)
    t = 1.0 / (1.0 + 0.3275911 * ax)
    poly = ((((a5 * t + a4) * t + a3) * t + a2) * t + a1) * t
    return sgn * (1.0 - poly * jnp.exp(-ax * ax))


def _node_body(Pn_ref, Pe_ref, tgt_ref,
               Wout_ref, bout_ref, Wg_ref, bg_ref, Wu_ref, bu_ref,
               Wd_ref, bd_ref, g1_ref, b1_ref, g2_ref, b2_ref,
               sc_ref, selt_ref, y_ref):
    f32 = jnp.float32
    numer = Pn_ref[...]
    s16 = Pe_ref[...]
    s16 = s16[0] + s16[1]
    sb = jnp.dot(s16, selt_ref[...], preferred_element_type=f32)
    att = numer / (sb + 1e-16)
    out = jnp.dot(att, Wout_ref[...], preferred_element_type=f32) + bout_ref[...]
    res_scale = sc_ref[0, 0]
    ffn_scale = sc_ref[0, 1]
    y = tgt_ref[...] + res_scale * out
    y = _ln_rows(y) * g1_ref[...] + b1_ref[...]
    gate = jnp.dot(y, Wg_ref[...], preferred_element_type=f32) + bg_ref[...]
    up = jnp.dot(y, Wu_ref[...], preferred_element_type=f32) + bu_ref[...]
    gelu = up * 0.5 * (1.0 + _erf(up * 0.7071067811865475))
    yff = jnp.dot(gate * gelu, Wd_ref[...], preferred_element_type=f32) + bd_ref[...]
    y = y + ffn_scale * yff
    y_ref[...] = _ln_rows(y) * g2_ref[...] + b2_ref[...]


def _tc_node(Pn, Pe, target, p):
    f32 = jnp.float32
    selt16 = (jnp.arange(ED)[:, None] == jnp.arange(D)[None, :] // DH).astype(f32)
    row = lambda v: v[None, :]
    scales = jnp.concatenate([p["res_scale"], p["ffn_scale"]])[None, :]
    weights = [
        p["Wout"], row(p["bout"]), p["Wg"], row(p["bg"]), p["Wu"], row(p["bu"]),
        p["Wd"], row(p["bd"]), row(p["g_1"]), row(p["b_1"]),
        row(p["g_2"]), row(p["b_2"]), scales, selt16,
    ]
    full = lambda a: pl.BlockSpec(a.shape, lambda i: (0,) * a.ndim)
    grid = N // BN
    return pl.pallas_call(
        _node_body,
        grid=(grid,),
        in_specs=(
            [pl.BlockSpec((BN, D), lambda i: (i, 0)),
             pl.BlockSpec((NC, BN, ED), lambda i: (0, i, 0)),
             pl.BlockSpec((BN, D), lambda i: (i, 0))]
            + [full(w) for w in weights]
        ),
        out_specs=pl.BlockSpec((BN, D), lambda i: (i, 0)),
        out_shape=jax.ShapeDtypeStruct((N, D), f32),
        compiler_params=pltpu.CompilerParams(
            dimension_semantics=("arbitrary",),
        ),
    )(Pn, Pe, target, *weights)


def kernel(source, target, edge_index, edge_attr, params):
    f32 = jnp.float32
    weights = _edge_weights(params)
    scatN = _make_sc_scatter(D, True)
    zrowN = jnp.zeros((8, D), f32)
    zrowE = jnp.zeros((8, ED), f32)
    Pn = jnp.zeros((N, D), f32)
    Pe = jnp.zeros((NC, N, ED), f32)
    ea_t = edge_attr.T

    def gather_c(c, tok):
        lo = c * EC
        sidx3 = lax.dynamic_slice_in_dim(edge_index[0], lo, EC).reshape(NW, NG, G)
        tidx3 = lax.dynamic_slice_in_dim(edge_index[1], lo, EC).reshape(NW, NG, G)
        return _sc_gather(source, target, sidx3, tidx3, tok)

    S = [None] * CH
    T = [None] * CH
    for c in range(CH):
        S[c], T[c] = gather_c(c, zrowN)
    upds = []
    for c in range(CH):
        lo = c * EC
        tidx = lax.dynamic_slice_in_dim(edge_index[1], lo, EC)
        numer, ex_pk, upd_t = _tc_edge(
            S[c], T[c], lax.dynamic_slice_in_dim(ea_t, lo, EC, axis=1), weights)
        Pn = scatN(numer, tidx, Pn, zrowN, zrowN)
        Pe = _sc_scatter_ex(ex_pk.reshape(EC, ED), tidx, Pe, zrowE)
        upds.append(upd_t)
    y = _tc_node(Pn, Pe, target, params)
    upd = jnp.concatenate(upds, axis=1).T
    return (y, upd)
